# Initial kernel scaffold; baseline (speedup 1.0000x reference)
import jax
import jax.numpy as jnp
from jax.experimental import pallas as pl


def kernel(x, batch, tpl_edge_index, euc_edge_index, W01, b01, g01, be01, W02, b02, g02, be02, W11, b11, g11, be11, W12, b12, g12, be12, Wm, bm, gm, bem):
    raise NotImplementedError("write your pallas kernel here")


# speedup vs baseline: 2.8229x; 2.8229x over previous
import jax
import jax.numpy as jnp
from jax import lax
from jax.experimental import pallas as pl
from jax.experimental.pallas import tpu as pltpu
from jax.experimental.pallas import tpu_sc as plsc

EPS = 1e-5
N = 10000
D = 128
H = 64
NT = 16
NPT = 640
NM = NT * NPT
MAGIC, MSHIFT = 6554, 22

EP = N + 320000
EP_PAD = 331776
PT = EP_PAD // NT
GCH = 256
NGCH = PT // GCH
SCH = 2048
NSCH = EP_PAD // SCH
EPH = EP_PAD // 2

_mesh = plsc.VectorSubcoreMesh(core_axis_name="c", subcore_axis_name="s")


def _node_mm_body(x_ref, w_ref, b_ref, o_ref):
    o_ref[0] = jnp.dot(x_ref[...], w_ref[0],
                       preferred_element_type=jnp.float32) + b_ref[0]


def _node_mm(x, wstack, bstack):
    return pl.pallas_call(
        _node_mm_body,
        grid=(2, 10),
        in_specs=[
            pl.BlockSpec((1000, D), lambda k, i: (i, 0)),
            pl.BlockSpec((1, D, D), lambda k, i: (k, 0, 0)),
            pl.BlockSpec((1, 1, D), lambda k, i: (k, 0, 0)),
        ],
        out_specs=pl.BlockSpec((1, 1000, D), lambda k, i: (k, i, 0)),
        out_shape=jax.ShapeDtypeStruct((2, N, D), jnp.float32),
    )(x, wstack, bstack)


def _edge_mlp_body(p_ref, w_ref, b_ref, o_ref):
    p = jnp.maximum(p_ref[0], 0.0)
    y = jnp.dot(p, w_ref[0], preferred_element_type=jnp.float32) + b_ref[0]
    o_ref[0] = jnp.maximum(y, 0.0)


def _edge_mlp(p2, w2bigs, b2bigs):
    return pl.pallas_call(
        _edge_mlp_body,
        grid=(2, EPH // 512),
        in_specs=[
            pl.BlockSpec((1, 512, 128), lambda k, i: (k, i, 0)),
            pl.BlockSpec((1, 128, 128), lambda k, i: (k, 0, 0)),
            pl.BlockSpec((1, 1, 128), lambda k, i: (k, 0, 0)),
        ],
        out_specs=pl.BlockSpec((1, 512, 128), lambda k, i: (k, i, 0)),
        out_shape=jax.ShapeDtypeStruct((2, EPH, 128), jnp.float32),
    )(p2, w2bigs, b2bigs)


def _final_body(m_ref, w_ref, b_ref, g_ref, e_ref, o_ref):
    z = (jnp.dot(m_ref[0], w_ref[:H], preferred_element_type=jnp.float32)
         + jnp.dot(m_ref[1], w_ref[H:], preferred_element_type=jnp.float32)
         + b_ref[0])
    o_ref[...] = jnp.maximum(z, 0.0) * g_ref[0] + e_ref[0]


def _final_mlp(m, wmf, bmf, gms, bem):
    return pl.pallas_call(
        _final_body,
        grid=(10,),
        in_specs=[
            pl.BlockSpec((2, 1000, H), lambda i: (0, i, 0)),
            pl.BlockSpec((2 * H, 2 * H), lambda i: (0, 0)),
            pl.BlockSpec((1, 2 * H), lambda i: (0, 0)),
            pl.BlockSpec((1, 2 * H), lambda i: (0, 0)),
            pl.BlockSpec((1, 2 * H), lambda i: (0, 0)),
        ],
        out_specs=pl.BlockSpec((1000, 2 * H), lambda i: (i, 0)),
        out_shape=jax.ShapeDtypeStruct((N, 2 * H), jnp.float32),
    )(m, wmf, bmf, gms, bem)


def _gather_body(t_hbm, dst_hbm, src_hbm, p_hbm,
                 idx_a, idx_b, buf_d, buf_s, buf_p, sem_a, sem_b):
    c = lax.axis_index("c")
    s = lax.axis_index("s")
    tile_base = s * PT

    def chunk(j, carry):
        base = pl.multiple_of(tile_base + j * GCH, GCH)
        pltpu.sync_copy(dst_hbm.at[c, pl.ds(base, GCH)], idx_a)
        pltpu.sync_copy(src_hbm.at[c, pl.ds(base, GCH)], idx_b)
        cps = []
        for h in range(2):
            cps.append(pltpu.async_copy(
                t_hbm.at[idx_a.at[pl.ds(h * 128, 128)]],
                buf_d.at[pl.ds(h * 128, 128)], sem_a))
            cps.append(pltpu.async_copy(
                t_hbm.at[idx_b.at[pl.ds(h * 128, 128)]],
                buf_s.at[pl.ds(h * 128, 128)], sem_b))
        for cp in cps:
            cp.wait()

        def addrow(rr, carry2):
            for u in range(2):
                for k in range(4):
                    buf_p[rr, pl.ds(u * H + k * 16, 16)] = (
                        buf_d[2 * rr + u, pl.ds(k * 16, 16)]
                        + buf_s[2 * rr + u, pl.ds(H + k * 16, 16)])
            return carry2
        lax.fori_loop(0, GCH // 2, addrow, 0)
        pltpu.sync_copy(buf_p, p_hbm.at[c, pl.ds(pl.multiple_of(base >> 1, GCH // 2), GCH // 2)])
        return carry
    lax.fori_loop(0, NGCH, chunk, 0)


def _edge_gather(t_flat, dst_adj, src_adj):
    f = pl.kernel(
        _gather_body,
        out_type=jax.ShapeDtypeStruct((2, EPH, 128), jnp.float32),
        mesh=_mesh,
        compiler_params=pltpu.CompilerParams(needs_layout_passes=False),
        scratch_types=[
            pltpu.VMEM((GCH,), jnp.int32),
            pltpu.VMEM((GCH,), jnp.int32),
            pltpu.VMEM((GCH, D), jnp.float32),
            pltpu.VMEM((GCH, D), jnp.float32),
            pltpu.VMEM((GCH // 2, D), jnp.float32),
            pltpu.SemaphoreType.DMA,
            pltpu.SemaphoreType.DMA,
        ],
    )
    return f(t_flat, dst_adj, src_adj)


_NEG = -3.0e38
ECAP = 2560
DB = 256


def _segmax_body(y_hbm, dst_hbm, m_hbm,
                 dchunk, elist, dlist, rows, table, sem_r0, sem_r1):
    c = lax.axis_index("c")
    s = lax.axis_index("s")
    iota = lax.iota(jnp.int32, 16)
    nodebase = s * NPT
    coff = c * N
    yhalf = c * EPH

    def initrow(r, carry):
        for k in range(4):
            table[r, pl.ds(k * 16, 16)] = jnp.full((16,), _NEG, jnp.float32)
        return carry
    lax.fori_loop(0, NPT + 1, initrow, 0)

    def initz(v, carry):
        elist[pl.ds(v * 16, 16)] = jnp.zeros((16,), jnp.int32)
        return carry
    lax.fori_loop(0, ECAP // 16, initz, 0)

    def drain_and_accum(off, ngroups):
        g0 = pltpu.async_copy(
            y_hbm.at[elist.at[pl.ds(off, 128)]], rows.at[pl.ds(0, 128)],
            sem_r0)
        g1 = pltpu.async_copy(
            y_hbm.at[elist.at[pl.ds(off + 128, 128)]],
            rows.at[pl.ds(128, 128)], sem_r1)
        g0.wait()
        g1.wait()

        def accum_group(g, carry):
            packs = dlist[pl.ds(off + g * 16, 16)]
            for u in range(16):
                p = packs[u]
                loff = p & 1023
                half = (p >> 10) * H
                for k in range(4):
                    tv = table[loff, pl.ds(k * 16, 16)]
                    rv = rows[g * 16 + u, pl.ds(half + k * 16, 16)]
                    table[loff, pl.ds(k * 16, 16)] = jnp.maximum(tv, rv)
            return carry
        lax.fori_loop(0, ngroups, accum_group, 0)

    def chunk(ci, w):
        cc = lax.rem(ci + s * 10, NSCH)
        pltpu.sync_copy(dst_hbm.at[c, pl.ds(pl.multiple_of(cc * SCH, SCH), SCH)], dchunk)

        def scan4(v, w2):
            G = 4
            ds_ = [dchunk[pl.ds((v * G + u) * 16, 16)] for u in range(G)]
            es = [(cc * SCH + (v * G + u) * 16) + iota for u in range(G)]
            ns = [dv - coff for dv in ds_]
            ms = [((((nv * MAGIC) >> MSHIFT) == s) & (ev < EP))
                  for nv, ev in zip(ns, es)]
            cnts = [jnp.max(plsc.all_reduce_population_count(mv))
                    for mv in ms]
            offs = [w2]
            for u in range(G - 1):
                offs.append(offs[-1] + cnts[u])
            for u in range(G):
                plsc.store_compressed(elist.at[pl.ds(offs[u], 16)],
                                      (es[u] >> 1) + yhalf, mask=ms[u])
                packed = (ns[u] - nodebase) + ((es[u] & 1) << 10)
                plsc.store_compressed(dlist.at[pl.ds(offs[u], 16)],
                                      packed, mask=ms[u])
            return offs[G - 1] + cnts[G - 1]
        w = lax.fori_loop(0, SCH // 64, scan4, w)

        nb = w >> 8

        def dr(k, carry):
            drain_and_accum(k * DB, DB // 16)
            return carry
        lax.fori_loop(0, nb, dr, 0)
        rem = w & (DB - 1)
        nbdb = w - rem

        def cp(t, carry):
            elist[pl.ds(t * 16, 16)] = elist[pl.ds(nbdb + t * 16, 16)]
            dlist[pl.ds(t * 16, 16)] = dlist[pl.ds(nbdb + t * 16, 16)]
            return carry
        lax.fori_loop(0, 16, cp, 0)
        return rem
    w = lax.fori_loop(0, NSCH, chunk, jnp.int32(0))

    plsc.store_compressed(dlist.at[pl.ds(w, 16)],
                          jnp.full((16,), NPT, jnp.int32),
                          mask=jnp.full((16,), True))
    drain_and_accum(0, (w + 15) >> 4)

    pltpu.sync_copy(table.at[pl.ds(0, NPT)],
                    m_hbm.at[c, pl.ds(pl.multiple_of(nodebase, 128), NPT)])


def _segmax(y2, dst_adj):
    f = pl.kernel(
        _segmax_body,
        out_type=jax.ShapeDtypeStruct((2, NM, H), jnp.float32),
        mesh=_mesh,
        compiler_params=pltpu.CompilerParams(needs_layout_passes=False),
        scratch_types=[
            pltpu.VMEM((SCH,), jnp.int32),
            pltpu.VMEM((ECAP,), jnp.int32),
            pltpu.VMEM((ECAP,), jnp.int32),
            pltpu.VMEM((DB, 2 * H), jnp.float32),
            pltpu.VMEM((NPT + 1, H), jnp.float32),
            pltpu.SemaphoreType.DMA,
            pltpu.SemaphoreType.DMA,
        ],
    )
    return f(y2, dst_adj)


def kernel(x, batch, tpl_edge_index, euc_edge_index,
           W01, b01, g01, be01, W02, b02, g02, be02,
           W11, b11, g11, be11, W12, b12, g12, be12,
           Wm, bm, gm, bem):
    s = (1.0 / jnp.sqrt(jnp.float32(1.0 + EPS))).astype(jnp.float32)
    lo = jnp.arange(N, dtype=jnp.int32)
    zpad = jnp.zeros((EP_PAD - EP,), jnp.int32)

    def prep(ei, cidx):
        dst = jnp.concatenate([ei[1], lo, zpad]) + (cidx * N)
        src = jnp.concatenate([ei[0], lo, zpad]) + (cidx * N)
        return dst, src

    dst0, src0 = prep(tpl_edge_index, 0)
    dst1, src1 = prep(euc_edge_index, 1)
    dst_adj = jnp.stack([dst0, dst1])
    src_adj = jnp.stack([src0, src1])

    zb = jnp.zeros((H,), jnp.float32)
    wcat0 = jnp.concatenate([W01[:D] - W01[D:], W01[D:]], axis=1)
    wcat1 = jnp.concatenate([W11[:D] - W11[D:], W11[D:]], axis=1)
    wstack = jnp.stack([wcat0, wcat1])
    bstack = jnp.stack([jnp.concatenate([b01, zb]),
                        jnp.concatenate([b11, zb])])[:, None, :]

    eye2 = jnp.eye(2, dtype=jnp.float32)

    def fold2(g1, be1, W2, b2):
        w2f = (g1 * s)[:, None] * W2
        b2f = be1 @ W2 + b2
        return jnp.kron(eye2, w2f), jnp.tile(b2f, 2)

    w2big0, b2big0 = fold2(g01, be01, W02, b02)
    w2big1, b2big1 = fold2(g11, be11, W12, b12)
    w2bigs = jnp.stack([w2big0, w2big1])
    b2bigs = jnp.stack([b2big0, b2big1])[:, None, :]

    scat = jnp.concatenate([g02 * s, g12 * s])
    becat = jnp.concatenate([be02, be12])
    wmf = scat[:, None] * Wm
    bmf = (becat @ Wm + bm)[None]
    gms = (gm * s)[None]
    bem2 = bem[None]

    t = _node_mm(x, wstack, bstack)
    t_flat = t.reshape(2 * N, D)
    p2 = _edge_gather(t_flat, dst_adj, src_adj)
    y2 = _edge_mlp(p2, w2bigs, b2bigs)
    y_flat = y2.reshape(2 * EPH, 128)
    m = _segmax(y_flat, dst_adj)
    return _final_mlp(m, wmf, bmf, gms, bem2)


# speedup vs baseline: 3.4179x; 1.2108x over previous
import jax
import jax.numpy as jnp
from jax import lax
from jax.experimental import pallas as pl
from jax.experimental.pallas import tpu as pltpu
from jax.experimental.pallas import tpu_sc as plsc

EPS = 1e-5
N = 10000
D = 128
H = 64
NT = 16
NPT = 640
NM = NT * NPT
MAGIC, MSHIFT = 6554, 22

EP = N + 320000
EP_PAD = 331776
PT = EP_PAD // NT
GCH = 128
NGCH = PT // GCH
SCH = 2048
NSCH = EP_PAD // SCH
EPH = EP_PAD // 2

_mesh = plsc.VectorSubcoreMesh(core_axis_name="c", subcore_axis_name="s")


def _node_mm_body(x_ref, w_ref, b_ref, o_ref):
    o_ref[0] = jnp.dot(x_ref[...], w_ref[0],
                       preferred_element_type=jnp.float32) + b_ref[0]


def _node_mm(x, wstack, bstack):
    return pl.pallas_call(
        _node_mm_body,
        grid=(2, 10),
        in_specs=[
            pl.BlockSpec((1000, D), lambda k, i: (i, 0)),
            pl.BlockSpec((1, D, D), lambda k, i: (k, 0, 0)),
            pl.BlockSpec((1, 1, D), lambda k, i: (k, 0, 0)),
        ],
        out_specs=pl.BlockSpec((1, 1000, D), lambda k, i: (k, i, 0)),
        out_shape=jax.ShapeDtypeStruct((2, N, D), jnp.float32),
    )(x, wstack, bstack)


def _edge_mlp_body(p_ref, w_ref, b_ref, o_ref):
    p = jnp.maximum(p_ref[0], 0.0)
    y = jnp.dot(p, w_ref[0], preferred_element_type=jnp.float32) + b_ref[0]
    o_ref[0] = jnp.maximum(y, 0.0)


def _edge_mlp(p2, w2bigs, b2bigs):
    return pl.pallas_call(
        _edge_mlp_body,
        grid=(2, EPH // 512),
        in_specs=[
            pl.BlockSpec((1, 512, 128), lambda k, i: (k, i, 0)),
            pl.BlockSpec((1, 128, 128), lambda k, i: (k, 0, 0)),
            pl.BlockSpec((1, 1, 128), lambda k, i: (k, 0, 0)),
        ],
        out_specs=pl.BlockSpec((1, 512, 128), lambda k, i: (k, i, 0)),
        out_shape=jax.ShapeDtypeStruct((2, EPH, 128), jnp.float32),
    )(p2, w2bigs, b2bigs)


def _final_body(m_ref, w_ref, b_ref, g_ref, e_ref, o_ref):
    z = (jnp.dot(m_ref[0], w_ref[:H], preferred_element_type=jnp.float32)
         + jnp.dot(m_ref[1], w_ref[H:], preferred_element_type=jnp.float32)
         + b_ref[0])
    o_ref[...] = jnp.maximum(z, 0.0) * g_ref[0] + e_ref[0]


def _final_mlp(m, wmf, bmf, gms, bem):
    return pl.pallas_call(
        _final_body,
        grid=(10,),
        in_specs=[
            pl.BlockSpec((2, 1000, H), lambda i: (0, i, 0)),
            pl.BlockSpec((2 * H, 2 * H), lambda i: (0, 0)),
            pl.BlockSpec((1, 2 * H), lambda i: (0, 0)),
            pl.BlockSpec((1, 2 * H), lambda i: (0, 0)),
            pl.BlockSpec((1, 2 * H), lambda i: (0, 0)),
        ],
        out_specs=pl.BlockSpec((1000, 2 * H), lambda i: (i, 0)),
        out_shape=jax.ShapeDtypeStruct((N, 2 * H), jnp.float32),
    )(m, wmf, bmf, gms, bem)


def _gather_body(t_hbm, dst_hbm, src_hbm, p_hbm,
                 idx_a, idx_b, buf_d, buf_s, buf_p,
                 sem_i0, sem_i1, sem_g0, sem_g1, sem_w0, sem_w1):
    c = lax.axis_index("c")
    s = lax.axis_index("s")
    tile_base = s * PT
    sem_i = (sem_i0, sem_i1)
    sem_g = (sem_g0, sem_g1)
    sem_w = (sem_w0, sem_w1)

    def fire_idx(p, j):
        base = pl.multiple_of(tile_base + j * GCH, GCH)
        pltpu.async_copy(dst_hbm.at[c, pl.ds(base, GCH)],
                         idx_a.at[p], sem_i[p])
        pltpu.async_copy(src_hbm.at[c, pl.ds(base, GCH)],
                         idx_b.at[p], sem_i[p])

    def wait_idx(p):
        pltpu.make_async_copy(dst_hbm.at[c, pl.ds(0, GCH)],
                              idx_a.at[p], sem_i[p]).wait()
        pltpu.make_async_copy(src_hbm.at[c, pl.ds(0, GCH)],
                              idx_b.at[p], sem_i[p]).wait()

    def fire_gather(p):
        pltpu.async_copy(t_hbm.at[idx_a.at[p]], buf_d.at[p], sem_g[p])
        pltpu.async_copy(t_hbm.at[idx_b.at[p]], buf_s.at[p], sem_g[p])

    def wait_gather(p):
        pltpu.make_async_copy(t_hbm.at[idx_a.at[p]], buf_d.at[p],
                              sem_g[p]).wait()
        pltpu.make_async_copy(t_hbm.at[idx_b.at[p]], buf_s.at[p],
                              sem_g[p]).wait()

    def fire_write(p, j):
        base = pl.multiple_of(tile_base + j * GCH, GCH)
        pltpu.async_copy(
            buf_p.at[p],
            p_hbm.at[c, pl.ds(pl.multiple_of(base >> 1, GCH // 2), GCH // 2)],
            sem_w[p])

    def wait_write(p):
        pltpu.make_async_copy(
            buf_p.at[p], p_hbm.at[c, pl.ds(0, GCH // 2)], sem_w[p]).wait()

    def process(p, j):
        wait_gather(p)
        fire_idx(p, j + 2)

        @pl.when(j >= 2)
        def _():
            wait_write(p)

        def addrow(rr, carry2):
            for u in range(2):
                for k in range(4):
                    buf_p[p, rr, pl.ds(u * H + k * 16, 16)] = (
                        buf_d[p, 2 * rr + u, pl.ds(k * 16, 16)]
                        + buf_s[p, 2 * rr + u, pl.ds(H + k * 16, 16)])
            return carry2
        lax.fori_loop(0, GCH // 2, addrow, 0)
        fire_write(p, j)
        wait_idx(p)
        fire_gather(p)

    for p in range(2):
        fire_idx(p, jnp.int32(p))
        wait_idx(p)
        fire_gather(p)

    def pair_body(t, carry):
        process(0, 2 * t)
        process(1, 2 * t + 1)
        return carry
    lax.fori_loop(0, NGCH // 2 - 1, pair_body, 0)

    for p, j in ((0, NGCH - 2), (1, NGCH - 1)):
        wait_gather(p)
        wait_write(p)

        def addrow2(rr, carry2, _p=p):
            for u in range(2):
                for k in range(4):
                    buf_p[_p, rr, pl.ds(u * H + k * 16, 16)] = (
                        buf_d[_p, 2 * rr + u, pl.ds(k * 16, 16)]
                        + buf_s[_p, 2 * rr + u, pl.ds(H + k * 16, 16)])
            return carry2
        lax.fori_loop(0, GCH // 2, addrow2, 0)
        fire_write(p, jnp.int32(j))
    wait_write(0)
    wait_write(1)


def _edge_gather(t_flat, dst_adj, src_adj):
    f = pl.kernel(
        _gather_body,
        out_type=jax.ShapeDtypeStruct((2, EPH, 128), jnp.float32),
        mesh=_mesh,
        compiler_params=pltpu.CompilerParams(needs_layout_passes=False),
        scratch_types=[
            pltpu.VMEM((2, GCH), jnp.int32),
            pltpu.VMEM((2, GCH), jnp.int32),
            pltpu.VMEM((2, GCH, D), jnp.float32),
            pltpu.VMEM((2, GCH, D), jnp.float32),
            pltpu.VMEM((2, GCH // 2, D), jnp.float32),
            pltpu.SemaphoreType.DMA,
            pltpu.SemaphoreType.DMA,
            pltpu.SemaphoreType.DMA,
            pltpu.SemaphoreType.DMA,
            pltpu.SemaphoreType.DMA,
            pltpu.SemaphoreType.DMA,
        ],
    )
    return f(t_flat, dst_adj, src_adj)


_NEG = -3.0e38
ECAP = 2560
DB = 256


def _segmax_body(y_hbm, dst_hbm, m_hbm,
                 dchunk, elist, dlist, rows, table, sem_r0, sem_r1):
    c = lax.axis_index("c")
    s = lax.axis_index("s")
    iota = lax.iota(jnp.int32, 16)
    nodebase = s * NPT
    coff = c * N
    yhalf = c * EPH

    def initrow(r, carry):
        for k in range(4):
            table[r, pl.ds(k * 16, 16)] = jnp.full((16,), _NEG, jnp.float32)
        return carry
    lax.fori_loop(0, NPT + 1, initrow, 0)

    def initz(v, carry):
        elist[pl.ds(v * 16, 16)] = jnp.zeros((16,), jnp.int32)
        return carry
    lax.fori_loop(0, ECAP // 16, initz, 0)

    def drain_and_accum(off, ngroups):
        g0 = pltpu.async_copy(
            y_hbm.at[elist.at[pl.ds(off, 128)]], rows.at[pl.ds(0, 128)],
            sem_r0)
        g1 = pltpu.async_copy(
            y_hbm.at[elist.at[pl.ds(off + 128, 128)]],
            rows.at[pl.ds(128, 128)], sem_r1)
        g0.wait()
        g1.wait()

        def accum_group(g, carry):
            packs = dlist[pl.ds(off + g * 16, 16)]
            for u in range(16):
                p = packs[u]
                loff = p & 1023
                half = (p >> 10) * H
                for k in range(4):
                    tv = table[loff, pl.ds(k * 16, 16)]
                    rv = rows[g * 16 + u, pl.ds(half + k * 16, 16)]
                    table[loff, pl.ds(k * 16, 16)] = jnp.maximum(tv, rv)
            return carry
        lax.fori_loop(0, ngroups, accum_group, 0)

    def chunk(ci, w):
        cc = lax.rem(ci + s * 10, NSCH)
        pltpu.sync_copy(dst_hbm.at[c, pl.ds(pl.multiple_of(cc * SCH, SCH), SCH)], dchunk)

        def scan4(v, w2):
            G = 4
            ds_ = [dchunk[pl.ds((v * G + u) * 16, 16)] for u in range(G)]
            es = [(cc * SCH + (v * G + u) * 16) + iota for u in range(G)]
            ns = [dv - coff for dv in ds_]
            ms = [((((nv * MAGIC) >> MSHIFT) == s) & (ev < EP))
                  for nv, ev in zip(ns, es)]
            cnts = [jnp.max(plsc.all_reduce_population_count(mv))
                    for mv in ms]
            offs = [w2]
            for u in range(G - 1):
                offs.append(offs[-1] + cnts[u])
            for u in range(G):
                plsc.store_compressed(elist.at[pl.ds(offs[u], 16)],
                                      (es[u] >> 1) + yhalf, mask=ms[u])
                packed = (ns[u] - nodebase) + ((es[u] & 1) << 10)
                plsc.store_compressed(dlist.at[pl.ds(offs[u], 16)],
                                      packed, mask=ms[u])
            return offs[G - 1] + cnts[G - 1]
        w = lax.fori_loop(0, SCH // 64, scan4, w)

        nb = w >> 8

        def dr(k, carry):
            drain_and_accum(k * DB, DB // 16)
            return carry
        lax.fori_loop(0, nb, dr, 0)
        rem = w & (DB - 1)
        nbdb = w - rem

        def cp(t, carry):
            elist[pl.ds(t * 16, 16)] = elist[pl.ds(nbdb + t * 16, 16)]
            dlist[pl.ds(t * 16, 16)] = dlist[pl.ds(nbdb + t * 16, 16)]
            return carry
        lax.fori_loop(0, 16, cp, 0)
        return rem
    w = lax.fori_loop(0, NSCH, chunk, jnp.int32(0))

    plsc.store_compressed(dlist.at[pl.ds(w, 16)],
                          jnp.full((16,), NPT, jnp.int32),
                          mask=jnp.full((16,), True))
    drain_and_accum(0, (w + 15) >> 4)

    pltpu.sync_copy(table.at[pl.ds(0, NPT)],
                    m_hbm.at[c, pl.ds(pl.multiple_of(nodebase, 128), NPT)])


def _segmax(y2, dst_adj):
    f = pl.kernel(
        _segmax_body,
        out_type=jax.ShapeDtypeStruct((2, NM, H), jnp.float32),
        mesh=_mesh,
        compiler_params=pltpu.CompilerParams(needs_layout_passes=False),
        scratch_types=[
            pltpu.VMEM((SCH,), jnp.int32),
            pltpu.VMEM((ECAP,), jnp.int32),
            pltpu.VMEM((ECAP,), jnp.int32),
            pltpu.VMEM((DB, 2 * H), jnp.float32),
            pltpu.VMEM((NPT + 1, H), jnp.float32),
            pltpu.SemaphoreType.DMA,
            pltpu.SemaphoreType.DMA,
        ],
    )
    return f(y2, dst_adj)


def kernel(x, batch, tpl_edge_index, euc_edge_index,
           W01, b01, g01, be01, W02, b02, g02, be02,
           W11, b11, g11, be11, W12, b12, g12, be12,
           Wm, bm, gm, bem):
    s = (1.0 / jnp.sqrt(jnp.float32(1.0 + EPS))).astype(jnp.float32)
    lo = jnp.arange(N, dtype=jnp.int32)
    zpad = jnp.zeros((EP_PAD - EP,), jnp.int32)

    def prep(ei, cidx):
        dst = jnp.concatenate([ei[1], lo, zpad]) + (cidx * N)
        src = jnp.concatenate([ei[0], lo, zpad]) + (cidx * N)
        return dst, src

    dst0, src0 = prep(tpl_edge_index, 0)
    dst1, src1 = prep(euc_edge_index, 1)
    dst_adj = jnp.stack([dst0, dst1])
    src_adj = jnp.stack([src0, src1])

    zb = jnp.zeros((H,), jnp.float32)
    wcat0 = jnp.concatenate([W01[:D] - W01[D:], W01[D:]], axis=1)
    wcat1 = jnp.concatenate([W11[:D] - W11[D:], W11[D:]], axis=1)
    wstack = jnp.stack([wcat0, wcat1])
    bstack = jnp.stack([jnp.concatenate([b01, zb]),
                        jnp.concatenate([b11, zb])])[:, None, :]

    eye2 = jnp.eye(2, dtype=jnp.float32)

    def fold2(g1, be1, W2, b2):
        w2f = (g1 * s)[:, None] * W2
        b2f = be1 @ W2 + b2
        return jnp.kron(eye2, w2f), jnp.tile(b2f, 2)

    w2big0, b2big0 = fold2(g01, be01, W02, b02)
    w2big1, b2big1 = fold2(g11, be11, W12, b12)
    w2bigs = jnp.stack([w2big0, w2big1])
    b2bigs = jnp.stack([b2big0, b2big1])[:, None, :]

    scat = jnp.concatenate([g02 * s, g12 * s])
    becat = jnp.concatenate([be02, be12])
    wmf = scat[:, None] * Wm
    bmf = (becat @ Wm + bm)[None]
    gms = (gm * s)[None]
    bem2 = bem[None]

    t = _node_mm(x, wstack, bstack)
    t_flat = t.reshape(2 * N, D)
    p2 = _edge_gather(t_flat, dst_adj, src_adj)
    y2 = _edge_mlp(p2, w2bigs, b2bigs)
    y_flat = y2.reshape(2 * EPH, 128)
    m = _segmax(y_flat, dst_adj)
    return _final_mlp(m, wmf, bmf, gms, bem2)


# speedup vs baseline: 3.6843x; 1.0779x over previous
import jax
import jax.numpy as jnp
from jax import lax
from jax.experimental import pallas as pl
from jax.experimental.pallas import tpu as pltpu
from jax.experimental.pallas import tpu_sc as plsc

EPS = 1e-5
N = 10000
D = 128
H = 64
NT = 16
NPT = 640
NM = NT * NPT
MAGIC, MSHIFT = 6554, 22

EP = N + 320000
EP_PAD = 331776
PT = EP_PAD // NT
GCH = 128
NGCH = PT // GCH
SCH = 2048
NSCH = EP_PAD // SCH
EPH = EP_PAD // 2

_mesh = plsc.VectorSubcoreMesh(core_axis_name="c", subcore_axis_name="s")


def _node_mm_body(x_ref, w_ref, b_ref, o_ref):
    o_ref[0] = jnp.dot(x_ref[...], w_ref[0],
                       preferred_element_type=jnp.float32) + b_ref[0]


def _node_mm(x, wstack, bstack):
    return pl.pallas_call(
        _node_mm_body,
        grid=(2, 10),
        in_specs=[
            pl.BlockSpec((1000, D), lambda k, i: (i, 0)),
            pl.BlockSpec((1, D, D), lambda k, i: (k, 0, 0)),
            pl.BlockSpec((1, 1, D), lambda k, i: (k, 0, 0)),
        ],
        out_specs=pl.BlockSpec((1, 1000, D), lambda k, i: (k, i, 0)),
        out_shape=jax.ShapeDtypeStruct((2, N, D), jnp.float32),
    )(x, wstack, bstack)


def _edge_mlp_body(p_ref, w_ref, b_ref, o_ref):
    p = jnp.maximum(p_ref[0], 0.0)
    y = jnp.dot(p, w_ref[0], preferred_element_type=jnp.float32) + b_ref[0]
    o_ref[0] = jnp.maximum(y, 0.0)


def _edge_mlp(p2, w2bigs, b2bigs):
    return pl.pallas_call(
        _edge_mlp_body,
        grid=(2, EPH // 512),
        in_specs=[
            pl.BlockSpec((1, 512, 128), lambda k, i: (k, i, 0)),
            pl.BlockSpec((1, 128, 128), lambda k, i: (k, 0, 0)),
            pl.BlockSpec((1, 1, 128), lambda k, i: (k, 0, 0)),
        ],
        out_specs=pl.BlockSpec((1, 512, 128), lambda k, i: (k, i, 0)),
        out_shape=jax.ShapeDtypeStruct((2, EPH, 128), jnp.float32),
    )(p2, w2bigs, b2bigs)


def _final_body(m_ref, w_ref, b_ref, g_ref, e_ref, o_ref):
    z = (jnp.dot(m_ref[0], w_ref[:H], preferred_element_type=jnp.float32)
         + jnp.dot(m_ref[1], w_ref[H:], preferred_element_type=jnp.float32)
         + b_ref[0])
    o_ref[...] = jnp.maximum(z, 0.0) * g_ref[0] + e_ref[0]


def _final_mlp(m, wmf, bmf, gms, bem):
    return pl.pallas_call(
        _final_body,
        grid=(10,),
        in_specs=[
            pl.BlockSpec((2, 1000, H), lambda i: (0, i, 0)),
            pl.BlockSpec((2 * H, 2 * H), lambda i: (0, 0)),
            pl.BlockSpec((1, 2 * H), lambda i: (0, 0)),
            pl.BlockSpec((1, 2 * H), lambda i: (0, 0)),
            pl.BlockSpec((1, 2 * H), lambda i: (0, 0)),
        ],
        out_specs=pl.BlockSpec((1000, 2 * H), lambda i: (i, 0)),
        out_shape=jax.ShapeDtypeStruct((N, 2 * H), jnp.float32),
    )(m, wmf, bmf, gms, bem)


def _gather_body(t_hbm, dst_hbm, src_hbm, p_hbm,
                 idx_a, idx_b, buf_d, buf_s, buf_p,
                 sem_i0, sem_i1, sem_g0, sem_g1, sem_w0, sem_w1):
    c = lax.axis_index("c")
    s = lax.axis_index("s")
    tile_base = s * PT
    sem_i = (sem_i0, sem_i1)
    sem_g = (sem_g0, sem_g1)
    sem_w = (sem_w0, sem_w1)

    def fire_idx(p, j):
        base = pl.multiple_of(tile_base + j * GCH, GCH)
        pltpu.async_copy(dst_hbm.at[c, pl.ds(base, GCH)],
                         idx_a.at[p], sem_i[p])
        pltpu.async_copy(src_hbm.at[c, pl.ds(base, GCH)],
                         idx_b.at[p], sem_i[p])

    def wait_idx(p):
        pltpu.make_async_copy(dst_hbm.at[c, pl.ds(0, GCH)],
                              idx_a.at[p], sem_i[p]).wait()
        pltpu.make_async_copy(src_hbm.at[c, pl.ds(0, GCH)],
                              idx_b.at[p], sem_i[p]).wait()

    def fire_gather(p):
        pltpu.async_copy(t_hbm.at[idx_a.at[p]], buf_d.at[p], sem_g[p])
        pltpu.async_copy(t_hbm.at[idx_b.at[p]], buf_s.at[p], sem_g[p])

    def wait_gather(p):
        pltpu.make_async_copy(t_hbm.at[idx_a.at[p]], buf_d.at[p],
                              sem_g[p]).wait()
        pltpu.make_async_copy(t_hbm.at[idx_b.at[p]], buf_s.at[p],
                              sem_g[p]).wait()

    def fire_write(p, j):
        base = pl.multiple_of(tile_base + j * GCH, GCH)
        pltpu.async_copy(
            buf_p.at[p],
            p_hbm.at[c, pl.ds(pl.multiple_of(base >> 1, GCH // 2), GCH // 2)],
            sem_w[p])

    def wait_write(p):
        pltpu.make_async_copy(
            buf_p.at[p], p_hbm.at[c, pl.ds(0, GCH // 2)], sem_w[p]).wait()

    def process(p, j):
        wait_gather(p)
        fire_idx(p, j + 2)

        @pl.when(j >= 2)
        def _():
            wait_write(p)

        def addrow(rr, carry2):
            for u in range(2):
                for k in range(4):
                    buf_p[p, rr, pl.ds(u * H + k * 16, 16)] = (
                        buf_d[p, 2 * rr + u, pl.ds(k * 16, 16)]
                        + buf_s[p, 2 * rr + u, pl.ds(H + k * 16, 16)])
            return carry2
        lax.fori_loop(0, GCH // 2, addrow, 0)
        fire_write(p, j)
        wait_idx(p)
        fire_gather(p)

    for p in range(2):
        fire_idx(p, jnp.int32(p))
        wait_idx(p)
        fire_gather(p)

    def pair_body(t, carry):
        process(0, 2 * t)
        process(1, 2 * t + 1)
        return carry
    lax.fori_loop(0, NGCH // 2 - 1, pair_body, 0)

    for p, j in ((0, NGCH - 2), (1, NGCH - 1)):
        wait_gather(p)
        wait_write(p)

        def addrow2(rr, carry2, _p=p):
            for u in range(2):
                for k in range(4):
                    buf_p[_p, rr, pl.ds(u * H + k * 16, 16)] = (
                        buf_d[_p, 2 * rr + u, pl.ds(k * 16, 16)]
                        + buf_s[_p, 2 * rr + u, pl.ds(H + k * 16, 16)])
            return carry2
        lax.fori_loop(0, GCH // 2, addrow2, 0)
        fire_write(p, jnp.int32(j))
    wait_write(0)
    wait_write(1)


def _edge_gather(t_flat, dst_adj, src_adj):
    f = pl.kernel(
        _gather_body,
        out_type=jax.ShapeDtypeStruct((2, EPH, 128), jnp.float32),
        mesh=_mesh,
        compiler_params=pltpu.CompilerParams(needs_layout_passes=False),
        scratch_types=[
            pltpu.VMEM((2, GCH), jnp.int32),
            pltpu.VMEM((2, GCH), jnp.int32),
            pltpu.VMEM((2, GCH, D), jnp.float32),
            pltpu.VMEM((2, GCH, D), jnp.float32),
            pltpu.VMEM((2, GCH // 2, D), jnp.float32),
            pltpu.SemaphoreType.DMA,
            pltpu.SemaphoreType.DMA,
            pltpu.SemaphoreType.DMA,
            pltpu.SemaphoreType.DMA,
            pltpu.SemaphoreType.DMA,
            pltpu.SemaphoreType.DMA,
        ],
    )
    return f(t_flat, dst_adj, src_adj)


_NEG = -3.0e38
ECAP = 2560
DB = 256


def _segmax_body(y_hbm, dst_hbm, m_hbm,
                 dchunk, elist, dlist, rows, table, sem_r0, sem_r1, sem_d):
    c = lax.axis_index("c")
    s = lax.axis_index("s")
    iota = lax.iota(jnp.int32, 16)
    nodebase = s * NPT
    coff = c * N
    yhalf = c * EPH

    def initrow(r, carry):
        for k in range(4):
            table[r, pl.ds(k * 16, 16)] = jnp.full((16,), _NEG, jnp.float32)
        return carry
    lax.fori_loop(0, NPT + 1, initrow, 0)

    def initz(v, carry):
        elist[pl.ds(v * 16, 16)] = jnp.zeros((16,), jnp.int32)
        return carry
    lax.fori_loop(0, ECAP // 16, initz, 0)

    def drain_and_accum(off, ngroups):
        g0 = pltpu.async_copy(
            y_hbm.at[elist.at[pl.ds(off, 128)]], rows.at[pl.ds(0, 128)],
            sem_r0)
        g1 = pltpu.async_copy(
            y_hbm.at[elist.at[pl.ds(off + 128, 128)]],
            rows.at[pl.ds(128, 128)], sem_r1)

        def accum_group(g, carry):
            packs = dlist[pl.ds(off + g * 16, 16)]
            for u in range(16):
                p = packs[u]
                loff = p & 1023
                half = (p >> 10) * H
                for k in range(4):
                    tv = table[loff, pl.ds(k * 16, 16)]
                    rv = rows[g * 16 + u, pl.ds(half + k * 16, 16)]
                    table[loff, pl.ds(k * 16, 16)] = jnp.maximum(tv, rv)
            return carry
        half_g = DB // 32
        g0.wait()
        first = jnp.minimum(ngroups, half_g)
        lax.fori_loop(0, first, accum_group, 0)
        g1.wait()
        lax.fori_loop(first, ngroups, accum_group, 0)

    def fire_dchunk(ci, p):
        cc = lax.rem(ci + s * 10, NSCH)
        pltpu.async_copy(
            dst_hbm.at[c, pl.ds(pl.multiple_of(cc * SCH, SCH), SCH)],
            dchunk.at[p], sem_d)

    def wait_dchunk(p):
        pltpu.make_async_copy(dst_hbm.at[c, pl.ds(0, SCH)],
                              dchunk.at[p], sem_d).wait()

    def chunk(ci, w):
        cc = lax.rem(ci + s * 10, NSCH)
        p_ = ci & 1
        wait_dchunk(p_)

        @pl.when(ci < NSCH - 1)
        def _():
            fire_dchunk(ci + 1, 1 - p_)

        def scan4(v, w2):
            G = 4
            ds_ = [dchunk[p_, pl.ds((v * G + u) * 16, 16)] for u in range(G)]
            es = [(cc * SCH + (v * G + u) * 16) + iota for u in range(G)]
            ns = [dv - coff for dv in ds_]
            ms = [((((nv * MAGIC) >> MSHIFT) == s) & (ev < EP))
                  for nv, ev in zip(ns, es)]
            cnts = [plsc.all_reduce_population_count(mv)[0] for mv in ms]
            offs = [w2]
            for u in range(G - 1):
                offs.append(offs[-1] + cnts[u])
            for u in range(G):
                plsc.store_compressed(elist.at[pl.ds(offs[u], 16)],
                                      (es[u] >> 1) + yhalf, mask=ms[u])
                packed = (ns[u] - nodebase) + ((es[u] & 1) << 10)
                plsc.store_compressed(dlist.at[pl.ds(offs[u], 16)],
                                      packed, mask=ms[u])
            return offs[G - 1] + cnts[G - 1]
        w = lax.fori_loop(0, SCH // 64, scan4, w)

        nb = w >> 8

        def dr(k, carry):
            drain_and_accum(k * DB, DB // 16)
            return carry
        lax.fori_loop(0, nb, dr, 0)
        rem = w & (DB - 1)
        nbdb = w - rem

        def cp(t, carry):
            elist[pl.ds(t * 16, 16)] = elist[pl.ds(nbdb + t * 16, 16)]
            dlist[pl.ds(t * 16, 16)] = dlist[pl.ds(nbdb + t * 16, 16)]
            return carry
        lax.fori_loop(0, 16, cp, 0)
        return rem
    fire_dchunk(jnp.int32(0), 0)
    w = lax.fori_loop(0, NSCH, chunk, jnp.int32(0))

    plsc.store_compressed(dlist.at[pl.ds(w, 16)],
                          jnp.full((16,), NPT, jnp.int32),
                          mask=jnp.full((16,), True))
    drain_and_accum(0, (w + 15) >> 4)

    pltpu.sync_copy(table.at[pl.ds(0, NPT)],
                    m_hbm.at[c, pl.ds(pl.multiple_of(nodebase, 128), NPT)])


def _segmax(y2, dst_adj):
    f = pl.kernel(
        _segmax_body,
        out_type=jax.ShapeDtypeStruct((2, NM, H), jnp.float32),
        mesh=_mesh,
        compiler_params=pltpu.CompilerParams(needs_layout_passes=False),
        scratch_types=[
            pltpu.VMEM((2, SCH), jnp.int32),
            pltpu.VMEM((ECAP,), jnp.int32),
            pltpu.VMEM((ECAP,), jnp.int32),
            pltpu.VMEM((DB, 2 * H), jnp.float32),
            pltpu.VMEM((NPT + 1, H), jnp.float32),
            pltpu.SemaphoreType.DMA,
            pltpu.SemaphoreType.DMA,
            pltpu.SemaphoreType.DMA,
        ],
    )
    return f(y2, dst_adj)


def kernel(x, batch, tpl_edge_index, euc_edge_index,
           W01, b01, g01, be01, W02, b02, g02, be02,
           W11, b11, g11, be11, W12, b12, g12, be12,
           Wm, bm, gm, bem):
    s = (1.0 / jnp.sqrt(jnp.float32(1.0 + EPS))).astype(jnp.float32)
    lo = jnp.arange(N, dtype=jnp.int32)
    zpad = jnp.zeros((EP_PAD - EP,), jnp.int32)

    def prep(ei, cidx):
        dst = jnp.concatenate([ei[1], lo, zpad]) + (cidx * N)
        src = jnp.concatenate([ei[0], lo, zpad]) + (cidx * N)
        return dst, src

    dst0, src0 = prep(tpl_edge_index, 0)
    dst1, src1 = prep(euc_edge_index, 1)
    dst_adj = jnp.stack([dst0, dst1])
    src_adj = jnp.stack([src0, src1])

    zb = jnp.zeros((H,), jnp.float32)
    wcat0 = jnp.concatenate([W01[:D] - W01[D:], W01[D:]], axis=1)
    wcat1 = jnp.concatenate([W11[:D] - W11[D:], W11[D:]], axis=1)
    wstack = jnp.stack([wcat0, wcat1])
    bstack = jnp.stack([jnp.concatenate([b01, zb]),
                        jnp.concatenate([b11, zb])])[:, None, :]

    eye2 = jnp.eye(2, dtype=jnp.float32)

    def fold2(g1, be1, W2, b2):
        w2f = (g1 * s)[:, None] * W2
        b2f = be1 @ W2 + b2
        return jnp.kron(eye2, w2f), jnp.tile(b2f, 2)

    w2big0, b2big0 = fold2(g01, be01, W02, b02)
    w2big1, b2big1 = fold2(g11, be11, W12, b12)
    w2bigs = jnp.stack([w2big0, w2big1])
    b2bigs = jnp.stack([b2big0, b2big1])[:, None, :]

    scat = jnp.concatenate([g02 * s, g12 * s])
    becat = jnp.concatenate([be02, be12])
    wmf = scat[:, None] * Wm
    bmf = (becat @ Wm + bm)[None]
    gms = (gm * s)[None]
    bem2 = bem[None]

    t = _node_mm(x, wstack, bstack)
    t_flat = t.reshape(2 * N, D)
    p2 = _edge_gather(t_flat, dst_adj, src_adj)
    y2 = _edge_mlp(p2, w2bigs, b2bigs)
    y_flat = y2.reshape(2 * EPH, 128)
    m = _segmax(y_flat, dst_adj)
    return _final_mlp(m, wmf, bmf, gms, bem2)


# speedup vs baseline: 3.6870x; 1.0007x over previous
import jax
import jax.numpy as jnp
from jax import lax
from jax.experimental import pallas as pl
from jax.experimental.pallas import tpu as pltpu
from jax.experimental.pallas import tpu_sc as plsc

EPS = 1e-5
N = 10000
D = 128
H = 64
NT = 16
NPT = 640
NM = NT * NPT
MAGIC, MSHIFT = 6554, 22

EP = N + 320000
EP_PAD = 331776
PT = EP_PAD // NT
GCH = 128
NGCH = PT // GCH
SCH = 2048
NSCH = EP_PAD // SCH
EPH = EP_PAD // 2

_mesh = plsc.VectorSubcoreMesh(core_axis_name="c", subcore_axis_name="s")


def _node_mm_body(x_ref, w_ref, b_ref, o_ref):
    o_ref[0] = jnp.dot(x_ref[...], w_ref[0],
                       preferred_element_type=jnp.float32) + b_ref[0]


def _node_mm(x, wstack, bstack):
    return pl.pallas_call(
        _node_mm_body,
        grid=(2, 10),
        in_specs=[
            pl.BlockSpec((1000, D), lambda k, i: (i, 0)),
            pl.BlockSpec((1, D, D), lambda k, i: (k, 0, 0)),
            pl.BlockSpec((1, 1, D), lambda k, i: (k, 0, 0)),
        ],
        out_specs=pl.BlockSpec((1, 1000, D), lambda k, i: (k, i, 0)),
        out_shape=jax.ShapeDtypeStruct((2, N, D), jnp.float32),
    )(x, wstack, bstack)


def _edge_mlp_body(p_ref, w_ref, b_ref, o_ref):
    p = jnp.maximum(p_ref[0], 0.0).astype(jnp.bfloat16)
    y = jnp.dot(p, w_ref[0], preferred_element_type=jnp.float32) + b_ref[0]
    o_ref[0] = jnp.maximum(y, 0.0)


def _edge_mlp(p2, w2bigs, b2bigs):
    return pl.pallas_call(
        _edge_mlp_body,
        grid=(2, EPH // 512),
        in_specs=[
            pl.BlockSpec((1, 512, 128), lambda k, i: (k, i, 0)),
            pl.BlockSpec((1, 128, 128), lambda k, i: (k, 0, 0)),
            pl.BlockSpec((1, 1, 128), lambda k, i: (k, 0, 0)),
        ],
        out_specs=pl.BlockSpec((1, 512, 128), lambda k, i: (k, i, 0)),
        out_shape=jax.ShapeDtypeStruct((2, EPH, 128), jnp.float32),
    )(p2, w2bigs, b2bigs)


def _final_body(m_ref, w_ref, b_ref, g_ref, e_ref, o_ref):
    z = (jnp.dot(m_ref[0], w_ref[:H], preferred_element_type=jnp.float32)
         + jnp.dot(m_ref[1], w_ref[H:], preferred_element_type=jnp.float32)
         + b_ref[0])
    o_ref[...] = jnp.maximum(z, 0.0) * g_ref[0] + e_ref[0]


def _final_mlp(m, wmf, bmf, gms, bem):
    return pl.pallas_call(
        _final_body,
        grid=(10,),
        in_specs=[
            pl.BlockSpec((2, 1000, H), lambda i: (0, i, 0)),
            pl.BlockSpec((2 * H, 2 * H), lambda i: (0, 0)),
            pl.BlockSpec((1, 2 * H), lambda i: (0, 0)),
            pl.BlockSpec((1, 2 * H), lambda i: (0, 0)),
            pl.BlockSpec((1, 2 * H), lambda i: (0, 0)),
        ],
        out_specs=pl.BlockSpec((1000, 2 * H), lambda i: (i, 0)),
        out_shape=jax.ShapeDtypeStruct((N, 2 * H), jnp.float32),
    )(m, wmf, bmf, gms, bem)


def _gather_body(t_hbm, dst_hbm, src_hbm, p_hbm,
                 idx_a, idx_b, buf_d, buf_s, buf_p, spm,
                 sem_i0, sem_i1, sem_g0, sem_g1, sem_w0, sem_w1):
    c = lax.axis_index("c")
    s = lax.axis_index("s")
    tile_base = s * PT
    sem_i = (sem_i0, sem_i1)
    sem_g = (sem_g0, sem_g1)
    sem_w = (sem_w0, sem_w1)


    def fire_idx(p, j):
        base = pl.multiple_of(tile_base + j * GCH, GCH)
        pltpu.async_copy(dst_hbm.at[c, pl.ds(base, GCH)],
                         idx_a.at[p], sem_i[p])
        pltpu.async_copy(src_hbm.at[c, pl.ds(base, GCH)],
                         idx_b.at[p], sem_i[p])

    def wait_idx(p):
        pltpu.make_async_copy(dst_hbm.at[c, pl.ds(0, GCH)],
                              idx_a.at[p], sem_i[p]).wait()
        pltpu.make_async_copy(src_hbm.at[c, pl.ds(0, GCH)],
                              idx_b.at[p], sem_i[p]).wait()

    def fire_gather(p):
        pltpu.async_copy(t_hbm.at[idx_a.at[p]], buf_d.at[p], sem_g[p])
        pltpu.async_copy(t_hbm.at[idx_b.at[p]], buf_s.at[p], sem_g[p])

    def wait_gather(p):
        pltpu.make_async_copy(t_hbm.at[idx_a.at[p]], buf_d.at[p],
                              sem_g[p]).wait()
        pltpu.make_async_copy(t_hbm.at[idx_b.at[p]], buf_s.at[p],
                              sem_g[p]).wait()

    def fire_write(p, j):
        base = pl.multiple_of(tile_base + j * GCH, GCH)
        pltpu.async_copy(
            buf_p.at[p],
            p_hbm.at[c, pl.ds(pl.multiple_of(base >> 1, GCH // 2), GCH // 2)],
            sem_w[p])

    def wait_write(p):
        pltpu.make_async_copy(
            buf_p.at[p], p_hbm.at[c, pl.ds(0, GCH // 2)], sem_w[p]).wait()

    def process(p, j):
        wait_gather(p)
        fire_idx(p, j + 2)

        @pl.when(j >= 2)
        def _():
            wait_write(p)

        def addrow(rr, carry2):
            for u in range(2):
                for k in range(4):
                    buf_p[p, rr, pl.ds(u * H + k * 16, 16)] = (
                        buf_d[p, 2 * rr + u, pl.ds(k * 16, 16)]
                        + buf_s[p, 2 * rr + u, pl.ds(H + k * 16, 16)])
            return carry2
        lax.fori_loop(0, GCH // 2, addrow, 0)
        fire_write(p, j)
        wait_idx(p)
        fire_gather(p)

    for p in range(2):
        fire_idx(p, jnp.int32(p))
        wait_idx(p)
        fire_gather(p)

    def pair_body(t, carry):
        process(0, 2 * t)
        process(1, 2 * t + 1)
        return carry
    lax.fori_loop(0, NGCH // 2 - 1, pair_body, 0)

    for p, j in ((0, NGCH - 2), (1, NGCH - 1)):
        wait_gather(p)
        wait_write(p)

        def addrow2(rr, carry2, _p=p):
            for u in range(2):
                for k in range(4):
                    buf_p[_p, rr, pl.ds(u * H + k * 16, 16)] = (
                        buf_d[_p, 2 * rr + u, pl.ds(k * 16, 16)]
                        + buf_s[_p, 2 * rr + u, pl.ds(H + k * 16, 16)])
            return carry2
        lax.fori_loop(0, GCH // 2, addrow2, 0)
        fire_write(p, jnp.int32(j))
    wait_write(0)
    wait_write(1)


def _edge_gather(t_flat, dst_adj, src_adj):
    f = pl.kernel(
        _gather_body,
        out_type=jax.ShapeDtypeStruct((2, EPH, 128), jnp.float32),
        mesh=_mesh,
        compiler_params=pltpu.CompilerParams(needs_layout_passes=False),
        scratch_types=[
            pltpu.VMEM((2, GCH), jnp.int32),
            pltpu.VMEM((2, GCH), jnp.int32),
            pltpu.VMEM((2, GCH, D), jnp.float32),
            pltpu.VMEM((2, GCH, D), jnp.float32),
            pltpu.VMEM((2, GCH // 2, D), jnp.float32),
            pltpu.VMEM_SHARED((16, D), jnp.float32),
            pltpu.SemaphoreType.DMA,
            pltpu.SemaphoreType.DMA,
            pltpu.SemaphoreType.DMA,
            pltpu.SemaphoreType.DMA,
            pltpu.SemaphoreType.DMA,
            pltpu.SemaphoreType.DMA,
        ],
    )
    return f(t_flat, dst_adj, src_adj)


_NEG = -3.0e38
ECAP = 2560
DB = 256


def _segmax_body(y_hbm, dst_hbm, m_hbm,
                 dchunk, elist, dlist, rows, table, sem_r0, sem_r1, sem_d):
    c = lax.axis_index("c")
    s = lax.axis_index("s")
    iota = lax.iota(jnp.int32, 16)
    nodebase = s * NPT
    coff = c * N
    yhalf = c * EPH

    def initrow(r, carry):
        for k in range(4):
            table[r, pl.ds(k * 16, 16)] = jnp.full((16,), _NEG, jnp.float32)
        return carry
    lax.fori_loop(0, NPT + 1, initrow, 0)

    def initz(v, carry):
        elist[pl.ds(v * 16, 16)] = jnp.zeros((16,), jnp.int32)
        return carry
    lax.fori_loop(0, ECAP // 16, initz, 0)

    def drain_and_accum(off, ngroups):
        g0 = pltpu.async_copy(
            y_hbm.at[elist.at[pl.ds(off, 128)]], rows.at[pl.ds(0, 128)],
            sem_r0)
        g1 = pltpu.async_copy(
            y_hbm.at[elist.at[pl.ds(off + 128, 128)]],
            rows.at[pl.ds(128, 128)], sem_r1)

        def accum_group(g, carry):
            packs = dlist[pl.ds(off + g * 16, 16)]
            for u in range(16):
                p = packs[u]
                loff = p & 1023
                half = (p >> 10) * H
                for k in range(4):
                    tv = table[loff, pl.ds(k * 16, 16)]
                    rv = rows[g * 16 + u, pl.ds(half + k * 16, 16)]
                    table[loff, pl.ds(k * 16, 16)] = jnp.maximum(tv, rv)
            return carry
        half_g = DB // 32
        g0.wait()
        first = jnp.minimum(ngroups, half_g)
        lax.fori_loop(0, first, accum_group, 0)
        g1.wait()
        lax.fori_loop(first, ngroups, accum_group, 0)

    def fire_dchunk(ci, p):
        cc = lax.rem(ci + s * 10, NSCH)
        pltpu.async_copy(
            dst_hbm.at[c, pl.ds(pl.multiple_of(cc * SCH, SCH), SCH)],
            dchunk.at[p], sem_d)

    def wait_dchunk(p):
        pltpu.make_async_copy(dst_hbm.at[c, pl.ds(0, SCH)],
                              dchunk.at[p], sem_d).wait()

    def chunk(ci, w):
        cc = lax.rem(ci + s * 10, NSCH)
        p_ = ci & 1
        wait_dchunk(p_)

        @pl.when(ci < NSCH - 1)
        def _():
            fire_dchunk(ci + 1, 1 - p_)

        def scan4(v, w2):
            G = 4
            ds_ = [dchunk[p_, pl.ds((v * G + u) * 16, 16)] for u in range(G)]
            es = [(cc * SCH + (v * G + u) * 16) + iota for u in range(G)]
            ns = [dv - coff for dv in ds_]
            ms = [((((nv * MAGIC) >> MSHIFT) == s) & (ev < EP))
                  for nv, ev in zip(ns, es)]
            cnts = [plsc.all_reduce_population_count(mv)[0] for mv in ms]
            offs = [w2]
            for u in range(G - 1):
                offs.append(offs[-1] + cnts[u])
            for u in range(G):
                plsc.store_compressed(elist.at[pl.ds(offs[u], 16)],
                                      (es[u] >> 1) + yhalf, mask=ms[u])
                packed = (ns[u] - nodebase) + ((es[u] & 1) << 10)
                plsc.store_compressed(dlist.at[pl.ds(offs[u], 16)],
                                      packed, mask=ms[u])
            return offs[G - 1] + cnts[G - 1]
        w = lax.fori_loop(0, SCH // 64, scan4, w)

        nb = w >> 8

        def dr(k, carry):
            drain_and_accum(k * DB, DB // 16)
            return carry
        lax.fori_loop(0, nb, dr, 0)
        rem = w & (DB - 1)
        nbdb = w - rem

        def cp(t, carry):
            elist[pl.ds(t * 16, 16)] = elist[pl.ds(nbdb + t * 16, 16)]
            dlist[pl.ds(t * 16, 16)] = dlist[pl.ds(nbdb + t * 16, 16)]
            return carry
        lax.fori_loop(0, 16, cp, 0)
        return rem
    fire_dchunk(jnp.int32(0), 0)
    w = lax.fori_loop(0, NSCH, chunk, jnp.int32(0))

    plsc.store_compressed(dlist.at[pl.ds(w, 16)],
                          jnp.full((16,), NPT, jnp.int32),
                          mask=jnp.full((16,), True))
    drain_and_accum(0, (w + 15) >> 4)

    pltpu.sync_copy(table.at[pl.ds(0, NPT)],
                    m_hbm.at[c, pl.ds(pl.multiple_of(nodebase, 128), NPT)])


def _segmax(y2, dst_adj):
    f = pl.kernel(
        _segmax_body,
        out_type=jax.ShapeDtypeStruct((2, NM, H), jnp.float32),
        mesh=_mesh,
        compiler_params=pltpu.CompilerParams(needs_layout_passes=False),
        scratch_types=[
            pltpu.VMEM((2, SCH), jnp.int32),
            pltpu.VMEM((ECAP,), jnp.int32),
            pltpu.VMEM((ECAP,), jnp.int32),
            pltpu.VMEM((DB, 2 * H), jnp.float32),
            pltpu.VMEM((NPT + 1, H), jnp.float32),
            pltpu.SemaphoreType.DMA,
            pltpu.SemaphoreType.DMA,
            pltpu.SemaphoreType.DMA,
        ],
    )
    return f(y2, dst_adj)


def kernel(x, batch, tpl_edge_index, euc_edge_index,
           W01, b01, g01, be01, W02, b02, g02, be02,
           W11, b11, g11, be11, W12, b12, g12, be12,
           Wm, bm, gm, bem):
    s = (1.0 / jnp.sqrt(jnp.float32(1.0 + EPS))).astype(jnp.float32)
    lo = jnp.arange(N, dtype=jnp.int32)
    zpad = jnp.zeros((EP_PAD - EP,), jnp.int32)

    def prep(ei, cidx):
        dst = jnp.concatenate([ei[1], lo, zpad]) + (cidx * N)
        src = jnp.concatenate([ei[0], lo, zpad]) + (cidx * N)
        return dst, src

    dst0, src0 = prep(tpl_edge_index, 0)
    dst1, src1 = prep(euc_edge_index, 1)
    dst_adj = jnp.stack([dst0, dst1])
    src_adj = jnp.stack([src0, src1])

    zb = jnp.zeros((H,), jnp.float32)
    wcat0 = jnp.concatenate([W01[:D] - W01[D:], W01[D:]], axis=1)
    wcat1 = jnp.concatenate([W11[:D] - W11[D:], W11[D:]], axis=1)
    wstack = jnp.stack([wcat0, wcat1])
    bstack = jnp.stack([jnp.concatenate([b01, zb]),
                        jnp.concatenate([b11, zb])])[:, None, :]

    eye2 = jnp.eye(2, dtype=jnp.float32)

    def fold2(g1, be1, W2, b2):
        w2f = (g1 * s)[:, None] * W2
        b2f = be1 @ W2 + b2
        return jnp.kron(eye2, w2f), jnp.tile(b2f, 2)

    w2big0, b2big0 = fold2(g01, be01, W02, b02)
    w2big1, b2big1 = fold2(g11, be11, W12, b12)
    w2bigs = jnp.stack([w2big0, w2big1]).astype(jnp.bfloat16)
    b2bigs = jnp.stack([b2big0, b2big1])[:, None, :]

    scat = jnp.concatenate([g02 * s, g12 * s])
    becat = jnp.concatenate([be02, be12])
    wmf = scat[:, None] * Wm
    bmf = (becat @ Wm + bm)[None]
    gms = (gm * s)[None]
    bem2 = bem[None]

    t = _node_mm(x, wstack, bstack)
    t_flat = t.reshape(2 * N, D)
    p2 = _edge_gather(t_flat, dst_adj, src_adj)
    y2 = _edge_mlp(p2, w2bigs, b2bigs)
    y_flat = y2.reshape(2 * EPH, 128)
    m = _segmax(y_flat, dst_adj)
    return _final_mlp(m, wmf, bmf, gms, bem2)


# speedup vs baseline: 3.8467x; 1.0433x over previous
import jax
import jax.numpy as jnp
from jax import lax
from jax.experimental import pallas as pl
from jax.experimental.pallas import tpu as pltpu
from jax.experimental.pallas import tpu_sc as plsc

EPS = 1e-5
N = 10000
D = 128
H = 64
NT = 16
NPT = 640
NM = NT * NPT
MAGIC, MSHIFT = 6554, 22

EP = N + 320000
EP_PAD = 331776
PT = EP_PAD // NT
GCH = 128
NGCH = PT // GCH
SCH = 2048
NSCH = EP_PAD // SCH
EPH = EP_PAD // 2

_mesh = plsc.VectorSubcoreMesh(core_axis_name="c", subcore_axis_name="s")


def _node_mm_body(x_ref, w_ref, b_ref, o_ref):
    o_ref[0] = jnp.dot(x_ref[...], w_ref[0],
                       preferred_element_type=jnp.float32) + b_ref[0]


def _node_mm(x, wstack, bstack):
    return pl.pallas_call(
        _node_mm_body,
        grid=(2, 10),
        in_specs=[
            pl.BlockSpec((1000, D), lambda k, i: (i, 0)),
            pl.BlockSpec((1, D, D), lambda k, i: (k, 0, 0)),
            pl.BlockSpec((1, 1, D), lambda k, i: (k, 0, 0)),
        ],
        out_specs=pl.BlockSpec((1, 1000, D), lambda k, i: (k, i, 0)),
        out_shape=jax.ShapeDtypeStruct((2, N, D), jnp.float32),
    )(x, wstack, bstack)


def _edge_mlp_body(p_ref, w_ref, b_ref, o_ref):
    p = jnp.maximum(p_ref[0], 0.0).astype(jnp.bfloat16)
    y = jnp.dot(p, w_ref[0], preferred_element_type=jnp.float32) + b_ref[0]
    o_ref[0] = jnp.maximum(y, 0.0)


def _edge_mlp(p2, w2bigs, b2bigs):
    return pl.pallas_call(
        _edge_mlp_body,
        grid=(2, EPH // 512),
        in_specs=[
            pl.BlockSpec((1, 512, 128), lambda k, i: (k, i, 0)),
            pl.BlockSpec((1, 128, 128), lambda k, i: (k, 0, 0)),
            pl.BlockSpec((1, 1, 128), lambda k, i: (k, 0, 0)),
        ],
        out_specs=pl.BlockSpec((1, 512, 128), lambda k, i: (k, i, 0)),
        out_shape=jax.ShapeDtypeStruct((2, EPH, 128), jnp.float32),
    )(p2, w2bigs, b2bigs)


def _final_body(m_ref, w_ref, b_ref, g_ref, e_ref, o_ref):
    z = (jnp.dot(m_ref[0], w_ref[:H], preferred_element_type=jnp.float32)
         + jnp.dot(m_ref[1], w_ref[H:], preferred_element_type=jnp.float32)
         + b_ref[0])
    o_ref[...] = jnp.maximum(z, 0.0) * g_ref[0] + e_ref[0]


def _final_mlp(m, wmf, bmf, gms, bem):
    return pl.pallas_call(
        _final_body,
        grid=(10,),
        in_specs=[
            pl.BlockSpec((2, 1000, H), lambda i: (0, i, 0)),
            pl.BlockSpec((2 * H, 2 * H), lambda i: (0, 0)),
            pl.BlockSpec((1, 2 * H), lambda i: (0, 0)),
            pl.BlockSpec((1, 2 * H), lambda i: (0, 0)),
            pl.BlockSpec((1, 2 * H), lambda i: (0, 0)),
        ],
        out_specs=pl.BlockSpec((1000, 2 * H), lambda i: (i, 0)),
        out_shape=jax.ShapeDtypeStruct((N, 2 * H), jnp.float32),
    )(m, wmf, bmf, gms, bem)


def _gather_body(t_hbm, dst_hbm, src_hbm, p_hbm,
                 idx_a, idx_b, buf_d, buf_s, buf_p, spm,
                 sem_i0, sem_i1, sem_g0, sem_g1, sem_w0, sem_w1):
    c = lax.axis_index("c")
    s = lax.axis_index("s")
    tile_base = s * PT
    sem_i = (sem_i0, sem_i1)
    sem_g = (sem_g0, sem_g1)
    sem_w = (sem_w0, sem_w1)


    def fire_idx(p, j):
        base = pl.multiple_of(tile_base + j * GCH, GCH)
        pltpu.async_copy(dst_hbm.at[c, pl.ds(base, GCH)],
                         idx_a.at[p], sem_i[p])
        pltpu.async_copy(src_hbm.at[c, pl.ds(base, GCH)],
                         idx_b.at[p], sem_i[p])

    def wait_idx(p):
        pltpu.make_async_copy(dst_hbm.at[c, pl.ds(0, GCH)],
                              idx_a.at[p], sem_i[p]).wait()
        pltpu.make_async_copy(src_hbm.at[c, pl.ds(0, GCH)],
                              idx_b.at[p], sem_i[p]).wait()

    def fire_gather(p):
        pltpu.async_copy(t_hbm.at[idx_a.at[p]], buf_d.at[p], sem_g[p])
        pltpu.async_copy(t_hbm.at[idx_b.at[p]], buf_s.at[p], sem_g[p])

    def wait_gather(p):
        pltpu.make_async_copy(t_hbm.at[idx_a.at[p]], buf_d.at[p],
                              sem_g[p]).wait()
        pltpu.make_async_copy(t_hbm.at[idx_b.at[p]], buf_s.at[p],
                              sem_g[p]).wait()

    def fire_write(p, j):
        base = pl.multiple_of(tile_base + j * GCH, GCH)
        pltpu.async_copy(
            buf_p.at[p],
            p_hbm.at[c, pl.ds(pl.multiple_of(base >> 1, GCH // 2), GCH // 2)],
            sem_w[p])

    def wait_write(p):
        pltpu.make_async_copy(
            buf_p.at[p], p_hbm.at[c, pl.ds(0, GCH // 2)], sem_w[p]).wait()

    def process(p, j):
        wait_gather(p)
        fire_idx(p, j + 2)

        @pl.when(j >= 2)
        def _():
            wait_write(p)

        def addrow(rr, carry2):
            for u in range(2):
                for k in range(4):
                    buf_p[p, rr, pl.ds(u * H + k * 16, 16)] = (
                        buf_d[p, 2 * rr + u, pl.ds(k * 16, 16)]
                        + buf_s[p, 2 * rr + u, pl.ds(H + k * 16, 16)])
            return carry2
        lax.fori_loop(0, GCH // 2, addrow, 0)
        fire_write(p, j)
        wait_idx(p)
        fire_gather(p)

    for p in range(2):
        fire_idx(p, jnp.int32(p))
        wait_idx(p)
        fire_gather(p)

    def pair_body(t, carry):
        process(0, 2 * t)
        process(1, 2 * t + 1)
        return carry
    lax.fori_loop(0, NGCH // 2 - 1, pair_body, 0)

    for p, j in ((0, NGCH - 2), (1, NGCH - 1)):
        wait_gather(p)
        wait_write(p)

        def addrow2(rr, carry2, _p=p):
            for u in range(2):
                for k in range(4):
                    buf_p[_p, rr, pl.ds(u * H + k * 16, 16)] = (
                        buf_d[_p, 2 * rr + u, pl.ds(k * 16, 16)]
                        + buf_s[_p, 2 * rr + u, pl.ds(H + k * 16, 16)])
            return carry2
        lax.fori_loop(0, GCH // 2, addrow2, 0)
        fire_write(p, jnp.int32(j))
    wait_write(0)
    wait_write(1)


def _edge_gather(t_flat, dst_adj, src_adj):
    f = pl.kernel(
        _gather_body,
        out_type=jax.ShapeDtypeStruct((2, EPH, 128), jnp.float32),
        mesh=_mesh,
        compiler_params=pltpu.CompilerParams(needs_layout_passes=False),
        scratch_types=[
            pltpu.VMEM((2, GCH), jnp.int32),
            pltpu.VMEM((2, GCH), jnp.int32),
            pltpu.VMEM((2, GCH, D), jnp.float32),
            pltpu.VMEM((2, GCH, D), jnp.float32),
            pltpu.VMEM((2, GCH // 2, D), jnp.float32),
            pltpu.VMEM_SHARED((16, D), jnp.float32),
            pltpu.SemaphoreType.DMA,
            pltpu.SemaphoreType.DMA,
            pltpu.SemaphoreType.DMA,
            pltpu.SemaphoreType.DMA,
            pltpu.SemaphoreType.DMA,
            pltpu.SemaphoreType.DMA,
        ],
    )
    return f(t_flat, dst_adj, src_adj)


_NEG = -3.0e38
ECAP = 2560
DB = 128


def _segmax_body(y_hbm, dst_hbm, m_hbm,
                 dchunk, elist, dlist, gidx, gpack, rows, table,
                 sem_r0, sem_r1, sem_d):
    c = lax.axis_index("c")
    s = lax.axis_index("s")
    iota = lax.iota(jnp.int32, 16)
    hiota = iota >> 1
    nodebase = s * NPT
    coff = c * N
    yhalf = c * EPH
    pvc = ((iota & 1) << 10) - (coff + nodebase)

    def initrow(r, carry):
        for k in range(4):
            table[r, pl.ds(k * 16, 16)] = jnp.full((16,), _NEG, jnp.float32)
        return carry
    lax.fori_loop(0, NPT + 1, initrow, 0)

    def initz(v, carry):
        elist[pl.ds(v * 16, 16)] = jnp.zeros((16,), jnp.int32)
        return carry
    lax.fori_loop(0, ECAP // 16, initz, 0)

    def accum_from(pack_ref, row_ref, ngroups):
        def accum_group(g, carry):
            packs = pack_ref[pl.ds(g * 16, 16)]
            for u in range(16):
                p = packs[u]
                loff = p & 1023
                half = (p >> 10) * H
                for k in range(4):
                    tv = table[loff, pl.ds(k * 16, 16)]
                    rv = row_ref[g * 16 + u, pl.ds(half + k * 16, 16)]
                    table[loff, pl.ds(k * 16, 16)] = jnp.maximum(tv, rv)
            return carry
        lax.fori_loop(0, ngroups, accum_group, 0)

    def sync_drain(off, ngroups):
        g0 = pltpu.async_copy(
            y_hbm.at[elist.at[pl.ds(off, DB // 2)]],
            rows.at[pl.ds(0, DB // 2)], sem_r0)
        g1 = pltpu.async_copy(
            y_hbm.at[elist.at[pl.ds(off + DB // 2, DB // 2)]],
            rows.at[pl.ds(DB // 2, DB // 2)], sem_r1)
        g0.wait()
        g1.wait()

        def accum_group(g, carry):
            packs = dlist[pl.ds(off + g * 16, 16)]
            for u in range(16):
                p = packs[u]
                loff = p & 1023
                half = (p >> 10) * H
                for k in range(4):
                    tv = table[loff, pl.ds(k * 16, 16)]
                    rv = rows[g * 16 + u, pl.ds(half + k * 16, 16)]
                    table[loff, pl.ds(k * 16, 16)] = jnp.maximum(tv, rv)
            return carry
        lax.fori_loop(0, ngroups, accum_group, 0)

    def wait_async_batch():
        pltpu.make_async_copy(y_hbm.at[gidx], rows, sem_r0).wait()

    def fire_async_batch():
        def cpb(t, carry):
            gidx[pl.ds(t * 16, 16)] = elist[pl.ds(t * 16, 16)]
            gpack[pl.ds(t * 16, 16)] = dlist[pl.ds(t * 16, 16)]
            return carry
        lax.fori_loop(0, DB // 16, cpb, 0)
        pltpu.async_copy(y_hbm.at[gidx], rows, sem_r0)

    def fire_dchunk(ci, p):
        cc = lax.rem(ci + s * 10, NSCH)
        pltpu.async_copy(
            dst_hbm.at[c, pl.ds(pl.multiple_of(cc * SCH, SCH), SCH)],
            dchunk.at[p], sem_d)

    def wait_dchunk(p):
        pltpu.make_async_copy(dst_hbm.at[c, pl.ds(0, SCH)],
                              dchunk.at[p], sem_d).wait()

    def chunk(ci, carry):
        w, infl = carry
        cc = lax.rem(ci + s * 10, NSCH)
        p_ = ci & 1
        wait_dchunk(p_)

        @pl.when(ci < NSCH - 1)
        def _():
            fire_dchunk(ci + 1, 1 - p_)

        def scan4(v, w2):
            G = 4
            ds_ = [dchunk[p_, pl.ds((v * G + u) * 16, 16)] for u in range(G)]
            bases = [cc * SCH + (v * G + u) * 16 for u in range(G)]
            ns = [dv - coff for dv in ds_]
            ms = [((((nv * MAGIC) >> MSHIFT) == s) & (iota < (EP - b)))
                  for nv, b in zip(ns, bases)]
            cnts = [plsc.all_reduce_population_count(mv)[0] for mv in ms]
            offs = [w2]
            for u in range(G - 1):
                offs.append(offs[-1] + cnts[u])
            for u in range(G):
                plsc.store_compressed(elist.at[pl.ds(offs[u], 16)],
                                      ((bases[u] >> 1) + yhalf) + hiota,
                                      mask=ms[u])
                plsc.store_compressed(dlist.at[pl.ds(offs[u], 16)],
                                      ds_[u] + pvc, mask=ms[u])
            return offs[G - 1] + cnts[G - 1]
        w = lax.fori_loop(0, SCH // 64, scan4, w)

        @pl.when(infl == 1)
        def _():
            wait_async_batch()
            accum_from(gpack, rows, DB // 16)

        nb = w >> 7

        def dr(k, carry2):
            sync_drain(k * DB, DB // 16)
            return carry2
        lax.fori_loop(1, nb, dr, 0)

        @pl.when(nb >= 1)
        def _():
            fire_async_batch()
        rem = w & (DB - 1)
        nbdb = w - rem

        def cp(t, carry2):
            elist[pl.ds(t * 16, 16)] = elist[pl.ds(nbdb + t * 16, 16)]
            dlist[pl.ds(t * 16, 16)] = dlist[pl.ds(nbdb + t * 16, 16)]
            return carry2
        lax.fori_loop(0, DB // 16, cp, 0)
        return (rem, (nb >= 1).astype(jnp.int32))
    fire_dchunk(jnp.int32(0), 0)
    w, infl = lax.fori_loop(0, NSCH, chunk, (jnp.int32(0), jnp.int32(0)))

    @pl.when(infl == 1)
    def _():
        wait_async_batch()
        accum_from(gpack, rows, DB // 16)

    plsc.store_compressed(dlist.at[pl.ds(w, 16)],
                          jnp.full((16,), NPT, jnp.int32),
                          mask=jnp.full((16,), True))
    sync_drain(0, (w + 15) >> 4)

    pltpu.sync_copy(table.at[pl.ds(0, NPT)],
                    m_hbm.at[c, pl.ds(pl.multiple_of(nodebase, 128), NPT)])


def _segmax(y2, dst_adj):
    f = pl.kernel(
        _segmax_body,
        out_type=jax.ShapeDtypeStruct((2, NM, H), jnp.float32),
        mesh=_mesh,
        compiler_params=pltpu.CompilerParams(needs_layout_passes=False),
        scratch_types=[
            pltpu.VMEM((2, SCH), jnp.int32),
            pltpu.VMEM((ECAP,), jnp.int32),
            pltpu.VMEM((ECAP,), jnp.int32),
            pltpu.VMEM((DB,), jnp.int32),
            pltpu.VMEM((DB,), jnp.int32),
            pltpu.VMEM((DB, 2 * H), jnp.float32),
            pltpu.VMEM((NPT + 1, H), jnp.float32),
            pltpu.SemaphoreType.DMA,
            pltpu.SemaphoreType.DMA,
            pltpu.SemaphoreType.DMA,
        ],
    )
    return f(y2, dst_adj)


def kernel(x, batch, tpl_edge_index, euc_edge_index,
           W01, b01, g01, be01, W02, b02, g02, be02,
           W11, b11, g11, be11, W12, b12, g12, be12,
           Wm, bm, gm, bem):
    s = (1.0 / jnp.sqrt(jnp.float32(1.0 + EPS))).astype(jnp.float32)
    lo = jnp.arange(N, dtype=jnp.int32)
    zpad = jnp.zeros((EP_PAD - EP,), jnp.int32)

    def prep(ei, cidx):
        dst = jnp.concatenate([ei[1], lo, zpad]) + (cidx * N)
        src = jnp.concatenate([ei[0], lo, zpad]) + (cidx * N)
        return dst, src

    dst0, src0 = prep(tpl_edge_index, 0)
    dst1, src1 = prep(euc_edge_index, 1)
    dst_adj = jnp.stack([dst0, dst1])
    src_adj = jnp.stack([src0, src1])

    zb = jnp.zeros((H,), jnp.float32)
    wcat0 = jnp.concatenate([W01[:D] - W01[D:], W01[D:]], axis=1)
    wcat1 = jnp.concatenate([W11[:D] - W11[D:], W11[D:]], axis=1)
    wstack = jnp.stack([wcat0, wcat1])
    bstack = jnp.stack([jnp.concatenate([b01, zb]),
                        jnp.concatenate([b11, zb])])[:, None, :]

    eye2 = jnp.eye(2, dtype=jnp.float32)

    def fold2(g1, be1, W2, b2):
        w2f = (g1 * s)[:, None] * W2
        b2f = be1 @ W2 + b2
        return jnp.kron(eye2, w2f), jnp.tile(b2f, 2)

    w2big0, b2big0 = fold2(g01, be01, W02, b02)
    w2big1, b2big1 = fold2(g11, be11, W12, b12)
    w2bigs = jnp.stack([w2big0, w2big1]).astype(jnp.bfloat16)
    b2bigs = jnp.stack([b2big0, b2big1])[:, None, :]

    scat = jnp.concatenate([g02 * s, g12 * s])
    becat = jnp.concatenate([be02, be12])
    wmf = scat[:, None] * Wm
    bmf = (becat @ Wm + bm)[None]
    gms = (gm * s)[None]
    bem2 = bem[None]

    t = _node_mm(x, wstack, bstack)
    t_flat = t.reshape(2 * N, D)
    p2 = _edge_gather(t_flat, dst_adj, src_adj)
    y2 = _edge_mlp(p2, w2bigs, b2bigs)
    y_flat = y2.reshape(2 * EPH, 128)
    m = _segmax(y_flat, dst_adj)
    return _final_mlp(m, wmf, bmf, gms, bem2)


# speedup vs baseline: 4.1700x; 1.0840x over previous
import jax
import jax.numpy as jnp
from jax import lax
from jax.experimental import pallas as pl
from jax.experimental.pallas import tpu as pltpu
from jax.experimental.pallas import tpu_sc as plsc

EPS = 1e-5
N = 10000
D = 128
H = 64
NT = 16
NPT = 640
NM = NT * NPT
MAGIC, MSHIFT = 6554, 22

EP = N + 320000
EP_PAD = 331776
PT = EP_PAD // NT
GCH = 128
NGCH = PT // GCH
SCH = 2048
NSCH = EP_PAD // SCH
EPH = EP_PAD // 2

_mesh = plsc.VectorSubcoreMesh(core_axis_name="c", subcore_axis_name="s")


def _node_mm_body(x_ref, w_ref, b_ref, o_ref):
    o_ref[0] = jnp.dot(x_ref[...], w_ref[0],
                       preferred_element_type=jnp.float32) + b_ref[0]


def _node_mm(x, wstack, bstack):
    return pl.pallas_call(
        _node_mm_body,
        grid=(2, 10),
        in_specs=[
            pl.BlockSpec((1000, D), lambda k, i: (i, 0)),
            pl.BlockSpec((1, D, D), lambda k, i: (k, 0, 0)),
            pl.BlockSpec((1, 1, D), lambda k, i: (k, 0, 0)),
        ],
        out_specs=pl.BlockSpec((1, 1000, D), lambda k, i: (k, i, 0)),
        out_shape=jax.ShapeDtypeStruct((2, N, D), jnp.float32),
    )(x, wstack, bstack)


def _edge_mlp_body(p_ref, w_ref, b_ref, o_ref):
    p = jnp.maximum(p_ref[0], 0.0).astype(jnp.bfloat16)
    y = jnp.dot(p, w_ref[0], preferred_element_type=jnp.float32) + b_ref[0]
    o_ref[0] = jnp.maximum(y, 0.0)


def _edge_mlp(p2, w2bigs, b2bigs):
    return pl.pallas_call(
        _edge_mlp_body,
        grid=(2, EPH // 512),
        in_specs=[
            pl.BlockSpec((1, 512, 128), lambda k, i: (k, i, 0)),
            pl.BlockSpec((1, 128, 128), lambda k, i: (k, 0, 0)),
            pl.BlockSpec((1, 1, 128), lambda k, i: (k, 0, 0)),
        ],
        out_specs=pl.BlockSpec((1, 512, 128), lambda k, i: (k, i, 0)),
        out_shape=jax.ShapeDtypeStruct((2, EPH, 128), jnp.float32),
    )(p2, w2bigs, b2bigs)


def _final_body(m_ref, w_ref, b_ref, g_ref, e_ref, o_ref):
    z = (jnp.dot(m_ref[0], w_ref[:H], preferred_element_type=jnp.float32)
         + jnp.dot(m_ref[1], w_ref[H:], preferred_element_type=jnp.float32)
         + b_ref[0])
    o_ref[...] = jnp.maximum(z, 0.0) * g_ref[0] + e_ref[0]


def _final_mlp(m, wmf, bmf, gms, bem):
    return pl.pallas_call(
        _final_body,
        grid=(10,),
        in_specs=[
            pl.BlockSpec((2, 1000, H), lambda i: (0, i, 0)),
            pl.BlockSpec((2 * H, 2 * H), lambda i: (0, 0)),
            pl.BlockSpec((1, 2 * H), lambda i: (0, 0)),
            pl.BlockSpec((1, 2 * H), lambda i: (0, 0)),
            pl.BlockSpec((1, 2 * H), lambda i: (0, 0)),
        ],
        out_specs=pl.BlockSpec((1000, 2 * H), lambda i: (i, 0)),
        out_shape=jax.ShapeDtypeStruct((N, 2 * H), jnp.float32),
    )(m, wmf, bmf, gms, bem)


def _gather_body(t_hbm, dst_hbm, src_hbm, p_hbm,
                 idx_a, idx_b, buf_d, buf_s, buf_p,
                 sem_i0, sem_i1, sem_g0, sem_g1, sem_w0, sem_w1):
    c = lax.axis_index("c")
    s = lax.axis_index("s")
    tile_base = s * PT
    sem_i = (sem_i0, sem_i1)
    sem_g = (sem_g0, sem_g1)
    sem_w = (sem_w0, sem_w1)


    def fire_idx(p, j):
        base = pl.multiple_of(tile_base + j * GCH, GCH)
        pltpu.async_copy(dst_hbm.at[c, pl.ds(base, GCH)],
                         idx_a.at[p], sem_i[p])
        pltpu.async_copy(src_hbm.at[c, pl.ds(base, GCH)],
                         idx_b.at[p], sem_i[p])

    def wait_idx(p):
        pltpu.make_async_copy(dst_hbm.at[c, pl.ds(0, GCH)],
                              idx_a.at[p], sem_i[p]).wait()
        pltpu.make_async_copy(src_hbm.at[c, pl.ds(0, GCH)],
                              idx_b.at[p], sem_i[p]).wait()

    def fire_gather(p):
        pltpu.async_copy(t_hbm.at[idx_a.at[p]], buf_d.at[p], sem_g[p])
        pltpu.async_copy(t_hbm.at[idx_b.at[p]], buf_s.at[p], sem_g[p])

    def wait_gather(p):
        pltpu.make_async_copy(t_hbm.at[idx_a.at[p]], buf_d.at[p],
                              sem_g[p]).wait()
        pltpu.make_async_copy(t_hbm.at[idx_b.at[p]], buf_s.at[p],
                              sem_g[p]).wait()

    def fire_write(p, j):
        base = pl.multiple_of(tile_base + j * GCH, GCH)
        pltpu.async_copy(
            buf_p.at[p],
            p_hbm.at[c, pl.ds(pl.multiple_of(base >> 1, GCH // 2), GCH // 2)],
            sem_w[p])

    def wait_write(p):
        pltpu.make_async_copy(
            buf_p.at[p], p_hbm.at[c, pl.ds(0, GCH // 2)], sem_w[p]).wait()

    def process(p, j):
        wait_gather(p)
        fire_idx(p, j + 2)

        @pl.when(j >= 2)
        def _():
            wait_write(p)

        @plsc.parallel_loop(0, GCH // 2, unroll=4)
        def addrow(rr):
            for u in range(2):
                for k in range(4):
                    buf_p[p, rr, pl.ds(u * H + k * 16, 16)] = (
                        buf_d[p, 2 * rr + u, pl.ds(k * 16, 16)]
                        + buf_s[p, 2 * rr + u, pl.ds(H + k * 16, 16)])
        fire_write(p, j)
        wait_idx(p)
        fire_gather(p)

    for p in range(2):
        fire_idx(p, jnp.int32(p))
        wait_idx(p)
        fire_gather(p)

    def pair_body(t, carry):
        process(0, 2 * t)
        process(1, 2 * t + 1)
        return carry
    lax.fori_loop(0, NGCH // 2 - 1, pair_body, 0)

    for p, j in ((0, NGCH - 2), (1, NGCH - 1)):
        wait_gather(p)
        wait_write(p)

        @plsc.parallel_loop(0, GCH // 2, unroll=4)
        def addrow2(rr, _p=p):
            for u in range(2):
                for k in range(4):
                    buf_p[_p, rr, pl.ds(u * H + k * 16, 16)] = (
                        buf_d[_p, 2 * rr + u, pl.ds(k * 16, 16)]
                        + buf_s[_p, 2 * rr + u, pl.ds(H + k * 16, 16)])
        fire_write(p, jnp.int32(j))
    wait_write(0)
    wait_write(1)


def _edge_gather(t_flat, dst_adj, src_adj):
    f = pl.kernel(
        _gather_body,
        out_type=jax.ShapeDtypeStruct((2, EPH, 128), jnp.float32),
        mesh=_mesh,
        compiler_params=pltpu.CompilerParams(needs_layout_passes=False),
        scratch_types=[
            pltpu.VMEM((2, GCH), jnp.int32),
            pltpu.VMEM((2, GCH), jnp.int32),
            pltpu.VMEM((2, GCH, D), jnp.float32),
            pltpu.VMEM((2, GCH, D), jnp.float32),
            pltpu.VMEM((2, GCH // 2, D), jnp.float32),
            pltpu.SemaphoreType.DMA,
            pltpu.SemaphoreType.DMA,
            pltpu.SemaphoreType.DMA,
            pltpu.SemaphoreType.DMA,
            pltpu.SemaphoreType.DMA,
            pltpu.SemaphoreType.DMA,
        ],
    )
    return f(t_flat, dst_adj, src_adj)


_NEG = -3.0e38
ECAP = 2560
DB = 128


def _segmax_body(y_hbm, dst_hbm, m_hbm,
                 dchunk, elist, dlist, gidx, gpack, rows, table,
                 sem_r0, sem_r1, sem_d):
    c = lax.axis_index("c")
    s = lax.axis_index("s")
    iota = lax.iota(jnp.int32, 16)
    hiota = iota >> 1
    nodebase = s * NPT
    coff = c * N
    yhalf = c * EPH
    pvc = ((iota & 1) << 10) - (coff + nodebase)

    def initrow(r, carry):
        for k in range(4):
            table[r, pl.ds(k * 16, 16)] = jnp.full((16,), _NEG, jnp.float32)
        return carry
    lax.fori_loop(0, NPT + 1, initrow, 0)

    def initz(v, carry):
        elist[pl.ds(v * 16, 16)] = jnp.zeros((16,), jnp.int32)
        return carry
    lax.fori_loop(0, ECAP // 16, initz, 0)

    def accum_from(pack_ref, row_ref, ngroups):
        def accum_group(g, carry):
            packs = pack_ref[pl.ds(g * 16, 16)]
            for u in range(16):
                p = packs[u]
                loff = p & 1023
                half = (p >> 10) * H
                for k in range(4):
                    tv = table[loff, pl.ds(k * 16, 16)]
                    rv = row_ref[g * 16 + u, pl.ds(half + k * 16, 16)]
                    table[loff, pl.ds(k * 16, 16)] = jnp.maximum(tv, rv)
            return carry
        lax.fori_loop(0, ngroups, accum_group, 0)

    def sync_drain(off, ngroups):
        g0 = pltpu.async_copy(
            y_hbm.at[elist.at[pl.ds(off, DB // 2)]],
            rows.at[pl.ds(0, DB // 2)], sem_r0)
        g1 = pltpu.async_copy(
            y_hbm.at[elist.at[pl.ds(off + DB // 2, DB // 2)]],
            rows.at[pl.ds(DB // 2, DB // 2)], sem_r1)
        g0.wait()
        g1.wait()

        def accum_group(g, carry):
            packs = dlist[pl.ds(off + g * 16, 16)]
            for u in range(16):
                p = packs[u]
                loff = p & 1023
                half = (p >> 10) * H
                for k in range(4):
                    tv = table[loff, pl.ds(k * 16, 16)]
                    rv = rows[g * 16 + u, pl.ds(half + k * 16, 16)]
                    table[loff, pl.ds(k * 16, 16)] = jnp.maximum(tv, rv)
            return carry
        lax.fori_loop(0, ngroups, accum_group, 0)

    def wait_async_batch():
        pltpu.make_async_copy(y_hbm.at[gidx], rows, sem_r0).wait()

    def fire_async_batch():
        def cpb(t, carry):
            gidx[pl.ds(t * 16, 16)] = elist[pl.ds(t * 16, 16)]
            gpack[pl.ds(t * 16, 16)] = dlist[pl.ds(t * 16, 16)]
            return carry
        lax.fori_loop(0, DB // 16, cpb, 0)
        pltpu.async_copy(y_hbm.at[gidx], rows, sem_r0)

    def fire_dchunk(ci, p):
        cc = lax.rem(ci + s * 10, NSCH)
        pltpu.async_copy(
            dst_hbm.at[c, pl.ds(pl.multiple_of(cc * SCH, SCH), SCH)],
            dchunk.at[p], sem_d)

    def wait_dchunk(p):
        pltpu.make_async_copy(dst_hbm.at[c, pl.ds(0, SCH)],
                              dchunk.at[p], sem_d).wait()

    def chunk(ci, carry):
        w, infl = carry
        cc = lax.rem(ci + s * 10, NSCH)
        p_ = ci & 1
        wait_dchunk(p_)

        @pl.when(ci < NSCH - 1)
        def _():
            fire_dchunk(ci + 1, 1 - p_)

        def scan4(v, w2):
            G = 4
            ds_ = [dchunk[p_, pl.ds((v * G + u) * 16, 16)] for u in range(G)]
            bases = [cc * SCH + (v * G + u) * 16 for u in range(G)]
            ns = [dv - coff for dv in ds_]
            ms = [((((nv * MAGIC) >> MSHIFT) == s) & (iota < (EP - b)))
                  for nv, b in zip(ns, bases)]
            cnts = [plsc.all_reduce_population_count(mv)[0] for mv in ms]
            offs = [w2]
            for u in range(G - 1):
                offs.append(offs[-1] + cnts[u])
            for u in range(G):
                plsc.store_compressed(elist.at[pl.ds(offs[u], 16)],
                                      ((bases[u] >> 1) + yhalf) + hiota,
                                      mask=ms[u])
                plsc.store_compressed(dlist.at[pl.ds(offs[u], 16)],
                                      ds_[u] + pvc, mask=ms[u])
            return offs[G - 1] + cnts[G - 1]
        w = lax.fori_loop(0, SCH // 64, scan4, w)

        @pl.when(infl == 1)
        def _():
            wait_async_batch()
            accum_from(gpack, rows, DB // 16)

        nb = w >> 7

        def dr(k, carry2):
            sync_drain(k * DB, DB // 16)
            return carry2
        lax.fori_loop(1, nb, dr, 0)

        @pl.when(nb >= 1)
        def _():
            fire_async_batch()
        rem = w & (DB - 1)
        nbdb = w - rem

        def cp(t, carry2):
            elist[pl.ds(t * 16, 16)] = elist[pl.ds(nbdb + t * 16, 16)]
            dlist[pl.ds(t * 16, 16)] = dlist[pl.ds(nbdb + t * 16, 16)]
            return carry2
        lax.fori_loop(0, DB // 16, cp, 0)
        return (rem, (nb >= 1).astype(jnp.int32))
    fire_dchunk(jnp.int32(0), 0)
    w, infl = lax.fori_loop(0, NSCH, chunk, (jnp.int32(0), jnp.int32(0)))

    @pl.when(infl == 1)
    def _():
        wait_async_batch()
        accum_from(gpack, rows, DB // 16)

    plsc.store_compressed(dlist.at[pl.ds(w, 16)],
                          jnp.full((16,), NPT, jnp.int32),
                          mask=jnp.full((16,), True))
    sync_drain(0, (w + 15) >> 4)

    pltpu.sync_copy(table.at[pl.ds(0, NPT)],
                    m_hbm.at[c, pl.ds(pl.multiple_of(nodebase, 128), NPT)])


def _segmax(y2, dst_adj):
    f = pl.kernel(
        _segmax_body,
        out_type=jax.ShapeDtypeStruct((2, NM, H), jnp.float32),
        mesh=_mesh,
        compiler_params=pltpu.CompilerParams(needs_layout_passes=False),
        scratch_types=[
            pltpu.VMEM((2, SCH), jnp.int32),
            pltpu.VMEM((ECAP,), jnp.int32),
            pltpu.VMEM((ECAP,), jnp.int32),
            pltpu.VMEM((DB,), jnp.int32),
            pltpu.VMEM((DB,), jnp.int32),
            pltpu.VMEM((DB, 2 * H), jnp.float32),
            pltpu.VMEM((NPT + 1, H), jnp.float32),
            pltpu.SemaphoreType.DMA,
            pltpu.SemaphoreType.DMA,
            pltpu.SemaphoreType.DMA,
        ],
    )
    return f(y2, dst_adj)


def kernel(x, batch, tpl_edge_index, euc_edge_index,
           W01, b01, g01, be01, W02, b02, g02, be02,
           W11, b11, g11, be11, W12, b12, g12, be12,
           Wm, bm, gm, bem):
    s = (1.0 / jnp.sqrt(jnp.float32(1.0 + EPS))).astype(jnp.float32)
    lo = jnp.arange(N, dtype=jnp.int32)
    zpad = jnp.zeros((EP_PAD - EP,), jnp.int32)

    def prep(ei, cidx):
        dst = jnp.concatenate([ei[1], lo, zpad]) + (cidx * N)
        src = jnp.concatenate([ei[0], lo, zpad]) + (cidx * N)
        return dst, src

    dst0, src0 = prep(tpl_edge_index, 0)
    dst1, src1 = prep(euc_edge_index, 1)
    dst_adj = jnp.stack([dst0, dst1])
    src_adj = jnp.stack([src0, src1])

    zb = jnp.zeros((H,), jnp.float32)
    wcat0 = jnp.concatenate([W01[:D] - W01[D:], W01[D:]], axis=1)
    wcat1 = jnp.concatenate([W11[:D] - W11[D:], W11[D:]], axis=1)
    wstack = jnp.stack([wcat0, wcat1])
    bstack = jnp.stack([jnp.concatenate([b01, zb]),
                        jnp.concatenate([b11, zb])])[:, None, :]

    eye2 = jnp.eye(2, dtype=jnp.float32)

    def fold2(g1, be1, W2, b2):
        w2f = (g1 * s)[:, None] * W2
        b2f = be1 @ W2 + b2
        return jnp.kron(eye2, w2f), jnp.tile(b2f, 2)

    w2big0, b2big0 = fold2(g01, be01, W02, b02)
    w2big1, b2big1 = fold2(g11, be11, W12, b12)
    w2bigs = jnp.stack([w2big0, w2big1]).astype(jnp.bfloat16)
    b2bigs = jnp.stack([b2big0, b2big1])[:, None, :]

    scat = jnp.concatenate([g02 * s, g12 * s])
    becat = jnp.concatenate([be02, be12])
    wmf = scat[:, None] * Wm
    bmf = (becat @ Wm + bm)[None]
    gms = (gm * s)[None]
    bem2 = bem[None]

    t = _node_mm(x, wstack, bstack)
    t_flat = t.reshape(2 * N, D)
    p2 = _edge_gather(t_flat, dst_adj, src_adj)
    y2 = _edge_mlp(p2, w2bigs, b2bigs)
    y_flat = y2.reshape(2 * EPH, 128)
    m = _segmax(y_flat, dst_adj)
    return _final_mlp(m, wmf, bmf, gms, bem2)


# speedup vs baseline: 4.9514x; 1.1874x over previous
import jax
import jax.numpy as jnp
from jax import lax
from jax.experimental import pallas as pl
from jax.experimental.pallas import tpu as pltpu
from jax.experimental.pallas import tpu_sc as plsc

EPS = 1e-5
N = 10000
D = 128
H = 64
NT = 16
NPT = 640
NM = NT * NPT
MAGIC, MSHIFT = 6554, 22

EP = N + 320000
EP_PAD = 331776
PT = EP_PAD // NT
GCH = 128
NGCH = PT // GCH
SCH = 2048
NSCH = EP_PAD // SCH
EPH = EP_PAD // 2

_mesh = plsc.VectorSubcoreMesh(core_axis_name="c", subcore_axis_name="s")


def _node_mm_body(x_ref, w_ref, b_ref, o_ref):
    o_ref[0] = jnp.dot(x_ref[...], w_ref[0],
                       preferred_element_type=jnp.float32) + b_ref[0]


def _node_mm(x, wstack, bstack):
    return pl.pallas_call(
        _node_mm_body,
        grid=(2, 10),
        in_specs=[
            pl.BlockSpec((1000, D), lambda k, i: (i, 0)),
            pl.BlockSpec((1, D, D), lambda k, i: (k, 0, 0)),
            pl.BlockSpec((1, 1, D), lambda k, i: (k, 0, 0)),
        ],
        out_specs=pl.BlockSpec((1, 1000, D), lambda k, i: (k, i, 0)),
        out_shape=jax.ShapeDtypeStruct((2, N, D), jnp.float32),
    )(x, wstack, bstack)


def _edge_mlp_body(p_ref, w_ref, b_ref, o_ref):
    p = jnp.maximum(p_ref[0], 0.0).astype(jnp.bfloat16)
    y = jnp.dot(p, w_ref[0], preferred_element_type=jnp.float32) + b_ref[0]
    o_ref[0] = jnp.maximum(y, 0.0)


def _edge_mlp(p2, w2bigs, b2bigs):
    return pl.pallas_call(
        _edge_mlp_body,
        grid=(2, EPH // 2048),
        in_specs=[
            pl.BlockSpec((1, 2048, 128), lambda k, i: (k, i, 0)),
            pl.BlockSpec((1, 128, 128), lambda k, i: (k, 0, 0)),
            pl.BlockSpec((1, 1, 128), lambda k, i: (k, 0, 0)),
        ],
        out_specs=pl.BlockSpec((1, 2048, 128), lambda k, i: (k, i, 0)),
        out_shape=jax.ShapeDtypeStruct((2, EPH, 128), jnp.float32),
    )(p2, w2bigs, b2bigs)


def _final_body(m_ref, w_ref, b_ref, g_ref, e_ref, o_ref):
    z = (jnp.dot(m_ref[0], w_ref[:H], preferred_element_type=jnp.float32)
         + jnp.dot(m_ref[1], w_ref[H:], preferred_element_type=jnp.float32)
         + b_ref[0])
    o_ref[...] = jnp.maximum(z, 0.0) * g_ref[0] + e_ref[0]


def _final_mlp(m, wmf, bmf, gms, bem):
    return pl.pallas_call(
        _final_body,
        grid=(10,),
        in_specs=[
            pl.BlockSpec((2, 1000, H), lambda i: (0, i, 0)),
            pl.BlockSpec((2 * H, 2 * H), lambda i: (0, 0)),
            pl.BlockSpec((1, 2 * H), lambda i: (0, 0)),
            pl.BlockSpec((1, 2 * H), lambda i: (0, 0)),
            pl.BlockSpec((1, 2 * H), lambda i: (0, 0)),
        ],
        out_specs=pl.BlockSpec((1000, 2 * H), lambda i: (i, 0)),
        out_shape=jax.ShapeDtypeStruct((N, 2 * H), jnp.float32),
    )(m, wmf, bmf, gms, bem)


def _gather_body(t_hbm, dst_hbm, src_hbm, p_hbm,
                 idx_a, idx_b, buf_d, buf_s, buf_p,
                 sem_i0, sem_i1, sem_g0, sem_g1, sem_w0, sem_w1):
    c = lax.axis_index("c")
    s = lax.axis_index("s")
    tile_base = s * PT
    sem_i = (sem_i0, sem_i1)
    sem_g = (sem_g0, sem_g1)
    sem_w = (sem_w0, sem_w1)


    def fire_idx(p, j):
        base = pl.multiple_of(tile_base + j * GCH, GCH)
        pltpu.async_copy(dst_hbm.at[c, pl.ds(base, GCH)],
                         idx_a.at[p], sem_i[p])
        pltpu.async_copy(src_hbm.at[c, pl.ds(base, GCH)],
                         idx_b.at[p], sem_i[p])

    def wait_idx(p):
        pltpu.make_async_copy(dst_hbm.at[c, pl.ds(0, GCH)],
                              idx_a.at[p], sem_i[p]).wait()
        pltpu.make_async_copy(src_hbm.at[c, pl.ds(0, GCH)],
                              idx_b.at[p], sem_i[p]).wait()

    def fire_gather(p):
        pltpu.async_copy(t_hbm.at[idx_a.at[p]], buf_d.at[p], sem_g[p])
        pltpu.async_copy(t_hbm.at[idx_b.at[p]], buf_s.at[p], sem_g[p])

    def wait_gather(p):
        pltpu.make_async_copy(t_hbm.at[idx_a.at[p]], buf_d.at[p],
                              sem_g[p]).wait()
        pltpu.make_async_copy(t_hbm.at[idx_b.at[p]], buf_s.at[p],
                              sem_g[p]).wait()

    def fire_write(p, j):
        base = pl.multiple_of(tile_base + j * GCH, GCH)
        pltpu.async_copy(
            buf_p.at[p],
            p_hbm.at[c, pl.ds(pl.multiple_of(base >> 1, GCH // 2), GCH // 2)],
            sem_w[p])

    def wait_write(p):
        pltpu.make_async_copy(
            buf_p.at[p], p_hbm.at[c, pl.ds(0, GCH // 2)], sem_w[p]).wait()

    def process(p, j):
        wait_gather(p)
        fire_idx(p, j + 2)

        @pl.when(j >= 2)
        def _():
            wait_write(p)

        @plsc.parallel_loop(0, GCH // 2, unroll=4)
        def addrow(rr):
            for u in range(2):
                for k in range(4):
                    buf_p[p, rr, pl.ds(u * H + k * 16, 16)] = (
                        buf_d[p, 2 * rr + u, pl.ds(k * 16, 16)]
                        + buf_s[p, 2 * rr + u, pl.ds(H + k * 16, 16)])
        fire_write(p, j)
        wait_idx(p)
        fire_gather(p)

    for p in range(2):
        fire_idx(p, jnp.int32(p))
        wait_idx(p)
        fire_gather(p)

    def pair_body(t, carry):
        process(0, 2 * t)
        process(1, 2 * t + 1)
        return carry
    lax.fori_loop(0, NGCH // 2 - 1, pair_body, 0)

    for p, j in ((0, NGCH - 2), (1, NGCH - 1)):
        wait_gather(p)
        wait_write(p)

        @plsc.parallel_loop(0, GCH // 2, unroll=4)
        def addrow2(rr, _p=p):
            for u in range(2):
                for k in range(4):
                    buf_p[_p, rr, pl.ds(u * H + k * 16, 16)] = (
                        buf_d[_p, 2 * rr + u, pl.ds(k * 16, 16)]
                        + buf_s[_p, 2 * rr + u, pl.ds(H + k * 16, 16)])
        fire_write(p, jnp.int32(j))
    wait_write(0)
    wait_write(1)


def _edge_gather(t_flat, dst_adj, src_adj):
    f = pl.kernel(
        _gather_body,
        out_type=jax.ShapeDtypeStruct((2, EPH, 128), jnp.float32),
        mesh=_mesh,
        compiler_params=pltpu.CompilerParams(needs_layout_passes=False),
        scratch_types=[
            pltpu.VMEM((2, GCH), jnp.int32),
            pltpu.VMEM((2, GCH), jnp.int32),
            pltpu.VMEM((2, GCH, D), jnp.float32),
            pltpu.VMEM((2, GCH, D), jnp.float32),
            pltpu.VMEM((2, GCH // 2, D), jnp.float32),
            pltpu.SemaphoreType.DMA,
            pltpu.SemaphoreType.DMA,
            pltpu.SemaphoreType.DMA,
            pltpu.SemaphoreType.DMA,
            pltpu.SemaphoreType.DMA,
            pltpu.SemaphoreType.DMA,
        ],
    )
    return f(t_flat, dst_adj, src_adj)


_NEG = -3.0e38
ECAP = 2560
DB = 128


def _segmax_body(y_hbm, dst_hbm, m_hbm,
                 dchunk, elist, dlist, gidx, gpack, rows, table,
                 sem_r0, sem_r1, sem_d):
    c = lax.axis_index("c")
    s = lax.axis_index("s")
    iota = lax.iota(jnp.int32, 16)
    hiota = iota >> 1
    nodebase = s * NPT
    coff = c * N
    yhalf = c * EPH
    pvc = ((iota & 1) << 10) - (coff + nodebase)

    def initrow(r, carry):
        for k in range(4):
            table[r, pl.ds(k * 16, 16)] = jnp.full((16,), _NEG, jnp.float32)
        return carry
    lax.fori_loop(0, NPT + 1, initrow, 0)

    def initz(v, carry):
        elist[pl.ds(v * 16, 16)] = jnp.zeros((16,), jnp.int32)
        return carry
    lax.fori_loop(0, ECAP // 16, initz, 0)

    def accum_from(pack_ref, row_ref, ngroups):
        def accum_group(g, carry):
            packs = pack_ref[pl.ds(g * 16, 16)]
            for u in range(16):
                p = packs[u]
                loff = p & 1023
                half = (p >> 10) * H
                for k in range(4):
                    tv = table[loff, pl.ds(k * 16, 16)]
                    rv = row_ref[g * 16 + u, pl.ds(half + k * 16, 16)]
                    table[loff, pl.ds(k * 16, 16)] = jnp.maximum(tv, rv)
            return carry
        lax.fori_loop(0, ngroups, accum_group, 0)

    def sync_drain(off, ngroups):
        g0 = pltpu.async_copy(
            y_hbm.at[elist.at[pl.ds(off, DB // 2)]],
            rows.at[pl.ds(0, DB // 2)], sem_r0)
        g1 = pltpu.async_copy(
            y_hbm.at[elist.at[pl.ds(off + DB // 2, DB // 2)]],
            rows.at[pl.ds(DB // 2, DB // 2)], sem_r1)
        g0.wait()
        g1.wait()

        def accum_group(g, carry):
            packs = dlist[pl.ds(off + g * 16, 16)]
            for u in range(16):
                p = packs[u]
                loff = p & 1023
                half = (p >> 10) * H
                for k in range(4):
                    tv = table[loff, pl.ds(k * 16, 16)]
                    rv = rows[g * 16 + u, pl.ds(half + k * 16, 16)]
                    table[loff, pl.ds(k * 16, 16)] = jnp.maximum(tv, rv)
            return carry
        lax.fori_loop(0, ngroups, accum_group, 0)

    def wait_async_batch():
        pltpu.make_async_copy(y_hbm.at[gidx], rows, sem_r0).wait()

    def fire_async_batch():
        def cpb(t, carry):
            gidx[pl.ds(t * 16, 16)] = elist[pl.ds(t * 16, 16)]
            gpack[pl.ds(t * 16, 16)] = dlist[pl.ds(t * 16, 16)]
            return carry
        lax.fori_loop(0, DB // 16, cpb, 0)
        pltpu.async_copy(y_hbm.at[gidx], rows, sem_r0)

    def fire_dchunk(ci, p):
        cc = lax.rem(ci + s * 10, NSCH)
        pltpu.async_copy(
            dst_hbm.at[c, pl.ds(pl.multiple_of(cc * SCH, SCH), SCH)],
            dchunk.at[p], sem_d)

    def wait_dchunk(p):
        pltpu.make_async_copy(dst_hbm.at[c, pl.ds(0, SCH)],
                              dchunk.at[p], sem_d).wait()

    def chunk(ci, carry):
        w, infl = carry
        cc = lax.rem(ci + s * 10, NSCH)
        p_ = ci & 1
        wait_dchunk(p_)

        @pl.when(ci < NSCH - 1)
        def _():
            fire_dchunk(ci + 1, 1 - p_)

        def scan4(v, w2):
            G = 4
            ds_ = [dchunk[p_, pl.ds((v * G + u) * 16, 16)] for u in range(G)]
            bases = [cc * SCH + (v * G + u) * 16 for u in range(G)]
            ns = [dv - coff for dv in ds_]
            ms = [((((nv * MAGIC) >> MSHIFT) == s) & (iota < (EP - b)))
                  for nv, b in zip(ns, bases)]
            cnts = [plsc.all_reduce_population_count(mv)[0] for mv in ms]
            offs = [w2]
            for u in range(G - 1):
                offs.append(offs[-1] + cnts[u])
            for u in range(G):
                plsc.store_compressed(elist.at[pl.ds(offs[u], 16)],
                                      ((bases[u] >> 1) + yhalf) + hiota,
                                      mask=ms[u])
                plsc.store_compressed(dlist.at[pl.ds(offs[u], 16)],
                                      ds_[u] + pvc, mask=ms[u])
            return offs[G - 1] + cnts[G - 1]
        w = lax.fori_loop(0, SCH // 64, scan4, w)

        @pl.when(infl == 1)
        def _():
            wait_async_batch()
            accum_from(gpack, rows, DB // 16)

        nb = w >> 7

        def dr(k, carry2):
            sync_drain(k * DB, DB // 16)
            return carry2
        lax.fori_loop(1, nb, dr, 0)

        @pl.when(nb >= 1)
        def _():
            fire_async_batch()
        rem = w & (DB - 1)
        nbdb = w - rem

        def cp(t, carry2):
            elist[pl.ds(t * 16, 16)] = elist[pl.ds(nbdb + t * 16, 16)]
            dlist[pl.ds(t * 16, 16)] = dlist[pl.ds(nbdb + t * 16, 16)]
            return carry2
        lax.fori_loop(0, DB // 16, cp, 0)
        return (rem, (nb >= 1).astype(jnp.int32))
    fire_dchunk(jnp.int32(0), 0)
    w, infl = lax.fori_loop(0, NSCH, chunk, (jnp.int32(0), jnp.int32(0)))

    @pl.when(infl == 1)
    def _():
        wait_async_batch()
        accum_from(gpack, rows, DB // 16)

    plsc.store_compressed(dlist.at[pl.ds(w, 16)],
                          jnp.full((16,), NPT, jnp.int32),
                          mask=jnp.full((16,), True))
    sync_drain(0, (w + 15) >> 4)

    pltpu.sync_copy(table.at[pl.ds(0, NPT)],
                    m_hbm.at[c, pl.ds(pl.multiple_of(nodebase, 128), NPT)])


def _segmax(y2, dst_adj):
    f = pl.kernel(
        _segmax_body,
        out_type=jax.ShapeDtypeStruct((2, NM, H), jnp.float32),
        mesh=_mesh,
        compiler_params=pltpu.CompilerParams(needs_layout_passes=False),
        scratch_types=[
            pltpu.VMEM((2, SCH), jnp.int32),
            pltpu.VMEM((ECAP,), jnp.int32),
            pltpu.VMEM((ECAP,), jnp.int32),
            pltpu.VMEM((DB,), jnp.int32),
            pltpu.VMEM((DB,), jnp.int32),
            pltpu.VMEM((DB, 2 * H), jnp.float32),
            pltpu.VMEM((NPT + 1, H), jnp.float32),
            pltpu.SemaphoreType.DMA,
            pltpu.SemaphoreType.DMA,
            pltpu.SemaphoreType.DMA,
        ],
    )
    return f(y2, dst_adj)


def kernel(x, batch, tpl_edge_index, euc_edge_index,
           W01, b01, g01, be01, W02, b02, g02, be02,
           W11, b11, g11, be11, W12, b12, g12, be12,
           Wm, bm, gm, bem):
    s = (1.0 / jnp.sqrt(jnp.float32(1.0 + EPS))).astype(jnp.float32)
    lo = jnp.arange(N, dtype=jnp.int32)
    zpad = jnp.zeros((EP_PAD - EP,), jnp.int32)

    def prep(ei, cidx):
        dst = jnp.concatenate([ei[1], lo, zpad]) + (cidx * N)
        src = jnp.concatenate([ei[0], lo, zpad]) + (cidx * N)
        return dst, src

    dst0, src0 = prep(tpl_edge_index, 0)
    dst1, src1 = prep(euc_edge_index, 1)
    dst_adj = jnp.stack([dst0, dst1])
    src_adj = jnp.stack([src0, src1])

    zb = jnp.zeros((H,), jnp.float32)
    wcat0 = jnp.concatenate([W01[:D] - W01[D:], W01[D:]], axis=1)
    wcat1 = jnp.concatenate([W11[:D] - W11[D:], W11[D:]], axis=1)
    wstack = jnp.stack([wcat0, wcat1])
    bstack = jnp.stack([jnp.concatenate([b01, zb]),
                        jnp.concatenate([b11, zb])])[:, None, :]

    eye2 = jnp.eye(2, dtype=jnp.float32)

    def fold2(g1, be1, W2, b2):
        w2f = (g1 * s)[:, None] * W2
        b2f = be1 @ W2 + b2
        return jnp.kron(eye2, w2f), jnp.tile(b2f, 2)

    w2big0, b2big0 = fold2(g01, be01, W02, b02)
    w2big1, b2big1 = fold2(g11, be11, W12, b12)
    w2bigs = jnp.stack([w2big0, w2big1]).astype(jnp.bfloat16)
    b2bigs = jnp.stack([b2big0, b2big1])[:, None, :]

    scat = jnp.concatenate([g02 * s, g12 * s])
    becat = jnp.concatenate([be02, be12])
    wmf = scat[:, None] * Wm
    bmf = (becat @ Wm + bm)[None]
    gms = (gm * s)[None]
    bem2 = bem[None]

    t = _node_mm(x, wstack, bstack)
    t_flat = t.reshape(2 * N, D)
    p2 = _edge_gather(t_flat, dst_adj, src_adj)
    y2 = _edge_mlp(p2, w2bigs, b2bigs)
    y_flat = y2.reshape(2 * EPH, 128)
    m = _segmax(y_flat, dst_adj)
    return _final_mlp(m, wmf, bmf, gms, bem2)


# speedup vs baseline: 4.9981x; 1.0094x over previous
import jax
import jax.numpy as jnp
from jax import lax
from jax.experimental import pallas as pl
from jax.experimental.pallas import tpu as pltpu
from jax.experimental.pallas import tpu_sc as plsc

EPS = 1e-5
N = 10000
D = 128
H = 64
NT = 16
NPT = 640
NM = NT * NPT
MAGIC, MSHIFT = 6554, 22

EP = N + 320000
EP_PAD = 331776
PT = EP_PAD // NT
GCH = 128
NGCH = PT // GCH
SCH = 2048
NSCH = EP_PAD // SCH
EPH = EP_PAD // 2

_mesh = plsc.VectorSubcoreMesh(core_axis_name="c", subcore_axis_name="s")


def _node_mm_body(x_ref, w_ref, b_ref, o_ref):
    o_ref[0] = jnp.dot(x_ref[...], w_ref[0],
                       preferred_element_type=jnp.float32) + b_ref[0]


def _node_mm(x, wstack, bstack):
    return pl.pallas_call(
        _node_mm_body,
        grid=(2, 2),
        in_specs=[
            pl.BlockSpec((5000, D), lambda k, i: (i, 0)),
            pl.BlockSpec((1, D, D), lambda k, i: (k, 0, 0)),
            pl.BlockSpec((1, 1, D), lambda k, i: (k, 0, 0)),
        ],
        out_specs=pl.BlockSpec((1, 5000, D), lambda k, i: (k, i, 0)),
        out_shape=jax.ShapeDtypeStruct((2, N, D), jnp.float32),
    )(x, wstack, bstack)


def _edge_mlp_body(p_ref, w_ref, b_ref, o_ref):
    p = jnp.maximum(p_ref[0], 0.0).astype(jnp.bfloat16)
    y = jnp.dot(p, w_ref[0], preferred_element_type=jnp.float32) + b_ref[0]
    o_ref[0] = jnp.maximum(y, 0.0)


def _edge_mlp(p2, w2bigs, b2bigs):
    return pl.pallas_call(
        _edge_mlp_body,
        grid=(2, EPH // 2048),
        in_specs=[
            pl.BlockSpec((1, 2048, 128), lambda k, i: (k, i, 0)),
            pl.BlockSpec((1, 128, 128), lambda k, i: (k, 0, 0)),
            pl.BlockSpec((1, 1, 128), lambda k, i: (k, 0, 0)),
        ],
        out_specs=pl.BlockSpec((1, 2048, 128), lambda k, i: (k, i, 0)),
        out_shape=jax.ShapeDtypeStruct((2, EPH, 128), jnp.float32),
    )(p2, w2bigs, b2bigs)


def _final_body(m_ref, w_ref, b_ref, g_ref, e_ref, o_ref):
    z = (jnp.dot(m_ref[0], w_ref[:H], preferred_element_type=jnp.float32)
         + jnp.dot(m_ref[1], w_ref[H:], preferred_element_type=jnp.float32)
         + b_ref[0])
    o_ref[...] = jnp.maximum(z, 0.0) * g_ref[0] + e_ref[0]


def _final_mlp(m, wmf, bmf, gms, bem):
    return pl.pallas_call(
        _final_body,
        grid=(2,),
        in_specs=[
            pl.BlockSpec((2, 5000, H), lambda i: (0, i, 0)),
            pl.BlockSpec((2 * H, 2 * H), lambda i: (0, 0)),
            pl.BlockSpec((1, 2 * H), lambda i: (0, 0)),
            pl.BlockSpec((1, 2 * H), lambda i: (0, 0)),
            pl.BlockSpec((1, 2 * H), lambda i: (0, 0)),
        ],
        out_specs=pl.BlockSpec((5000, 2 * H), lambda i: (i, 0)),
        out_shape=jax.ShapeDtypeStruct((N, 2 * H), jnp.float32),
    )(m, wmf, bmf, gms, bem)


def _gather_body(t_hbm, dst_hbm, src_hbm, p_hbm,
                 idx_a, idx_b, buf_d, buf_s, buf_p,
                 sem_i0, sem_i1, sem_g0, sem_g1, sem_w0, sem_w1):
    c = lax.axis_index("c")
    s = lax.axis_index("s")
    tile_base = s * PT
    sem_i = (sem_i0, sem_i1)
    sem_g = (sem_g0, sem_g1)
    sem_w = (sem_w0, sem_w1)


    def fire_idx(p, j):
        base = pl.multiple_of(tile_base + j * GCH, GCH)
        pltpu.async_copy(dst_hbm.at[c, pl.ds(base, GCH)],
                         idx_a.at[p], sem_i[p])
        pltpu.async_copy(src_hbm.at[c, pl.ds(base, GCH)],
                         idx_b.at[p], sem_i[p])

    def wait_idx(p):
        pltpu.make_async_copy(dst_hbm.at[c, pl.ds(0, GCH)],
                              idx_a.at[p], sem_i[p]).wait()
        pltpu.make_async_copy(src_hbm.at[c, pl.ds(0, GCH)],
                              idx_b.at[p], sem_i[p]).wait()

    def fire_gather(p):
        pltpu.async_copy(t_hbm.at[idx_a.at[p]], buf_d.at[p], sem_g[p])
        pltpu.async_copy(t_hbm.at[idx_b.at[p]], buf_s.at[p], sem_g[p])

    def wait_gather(p):
        pltpu.make_async_copy(t_hbm.at[idx_a.at[p]], buf_d.at[p],
                              sem_g[p]).wait()
        pltpu.make_async_copy(t_hbm.at[idx_b.at[p]], buf_s.at[p],
                              sem_g[p]).wait()

    def fire_write(p, j):
        base = pl.multiple_of(tile_base + j * GCH, GCH)
        pltpu.async_copy(
            buf_p.at[p],
            p_hbm.at[c, pl.ds(pl.multiple_of(base >> 1, GCH // 2), GCH // 2)],
            sem_w[p])

    def wait_write(p):
        pltpu.make_async_copy(
            buf_p.at[p], p_hbm.at[c, pl.ds(0, GCH // 2)], sem_w[p]).wait()

    def process(p, j):
        wait_gather(p)
        fire_idx(p, j + 2)

        @pl.when(j >= 2)
        def _():
            wait_write(p)

        @plsc.parallel_loop(0, GCH // 2, unroll=4)
        def addrow(rr):
            for u in range(2):
                for k in range(4):
                    buf_p[p, rr, pl.ds(u * H + k * 16, 16)] = (
                        buf_d[p, 2 * rr + u, pl.ds(k * 16, 16)]
                        + buf_s[p, 2 * rr + u, pl.ds(H + k * 16, 16)])
        fire_write(p, j)
        wait_idx(p)
        fire_gather(p)

    for p in range(2):
        fire_idx(p, jnp.int32(p))
        wait_idx(p)
        fire_gather(p)

    def pair_body(t, carry):
        process(0, 2 * t)
        process(1, 2 * t + 1)
        return carry
    lax.fori_loop(0, NGCH // 2 - 1, pair_body, 0)

    for p, j in ((0, NGCH - 2), (1, NGCH - 1)):
        wait_gather(p)
        wait_write(p)

        @plsc.parallel_loop(0, GCH // 2, unroll=4)
        def addrow2(rr, _p=p):
            for u in range(2):
                for k in range(4):
                    buf_p[_p, rr, pl.ds(u * H + k * 16, 16)] = (
                        buf_d[_p, 2 * rr + u, pl.ds(k * 16, 16)]
                        + buf_s[_p, 2 * rr + u, pl.ds(H + k * 16, 16)])
        fire_write(p, jnp.int32(j))
    wait_write(0)
    wait_write(1)


def _edge_gather(t_flat, dst_adj, src_adj):
    f = pl.kernel(
        _gather_body,
        out_type=jax.ShapeDtypeStruct((2, EPH, 128), jnp.float32),
        mesh=_mesh,
        compiler_params=pltpu.CompilerParams(needs_layout_passes=False),
        scratch_types=[
            pltpu.VMEM((2, GCH), jnp.int32),
            pltpu.VMEM((2, GCH), jnp.int32),
            pltpu.VMEM((2, GCH, D), jnp.float32),
            pltpu.VMEM((2, GCH, D), jnp.float32),
            pltpu.VMEM((2, GCH // 2, D), jnp.float32),
            pltpu.SemaphoreType.DMA,
            pltpu.SemaphoreType.DMA,
            pltpu.SemaphoreType.DMA,
            pltpu.SemaphoreType.DMA,
            pltpu.SemaphoreType.DMA,
            pltpu.SemaphoreType.DMA,
        ],
    )
    return f(t_flat, dst_adj, src_adj)


_NEG = -3.0e38
ECAP = 2560
DB = 128


def _segmax_body(y_hbm, dst_hbm, m_hbm,
                 dchunk, elist, dlist, gidx, gpack, rows, table,
                 sem_r0, sem_r1, sem_d):
    c = lax.axis_index("c")
    s = lax.axis_index("s")
    iota = lax.iota(jnp.int32, 16)
    hiota = iota >> 1
    nodebase = s * NPT
    coff = c * N
    yhalf = c * EPH
    pvc = ((iota & 1) << 10) - (coff + nodebase)

    def initrow(r, carry):
        for k in range(4):
            table[r, pl.ds(k * 16, 16)] = jnp.full((16,), _NEG, jnp.float32)
        return carry
    lax.fori_loop(0, NPT + 1, initrow, 0)

    def initz(v, carry):
        elist[pl.ds(v * 16, 16)] = jnp.zeros((16,), jnp.int32)
        return carry
    lax.fori_loop(0, ECAP // 16, initz, 0)

    def accum_from(pack_ref, row_ref, ngroups):
        def accum_group(g, carry):
            packs = pack_ref[pl.ds(g * 16, 16)]
            for u in range(16):
                p = packs[u]
                loff = p & 1023
                half = (p >> 10) * H
                for k in range(4):
                    tv = table[loff, pl.ds(k * 16, 16)]
                    rv = row_ref[g * 16 + u, pl.ds(half + k * 16, 16)]
                    table[loff, pl.ds(k * 16, 16)] = jnp.maximum(tv, rv)
            return carry
        lax.fori_loop(0, ngroups, accum_group, 0)

    def sync_drain(off, ngroups):
        g0 = pltpu.async_copy(
            y_hbm.at[elist.at[pl.ds(off, DB // 2)]],
            rows.at[pl.ds(0, DB // 2)], sem_r0)
        g1 = pltpu.async_copy(
            y_hbm.at[elist.at[pl.ds(off + DB // 2, DB // 2)]],
            rows.at[pl.ds(DB // 2, DB // 2)], sem_r1)
        g0.wait()
        g1.wait()

        def accum_group(g, carry):
            packs = dlist[pl.ds(off + g * 16, 16)]
            for u in range(16):
                p = packs[u]
                loff = p & 1023
                half = (p >> 10) * H
                for k in range(4):
                    tv = table[loff, pl.ds(k * 16, 16)]
                    rv = rows[g * 16 + u, pl.ds(half + k * 16, 16)]
                    table[loff, pl.ds(k * 16, 16)] = jnp.maximum(tv, rv)
            return carry
        lax.fori_loop(0, ngroups, accum_group, 0)

    def wait_async_batch():
        pltpu.make_async_copy(y_hbm.at[gidx], rows, sem_r0).wait()

    def fire_async_batch():
        def cpb(t, carry):
            gidx[pl.ds(t * 16, 16)] = elist[pl.ds(t * 16, 16)]
            gpack[pl.ds(t * 16, 16)] = dlist[pl.ds(t * 16, 16)]
            return carry
        lax.fori_loop(0, DB // 16, cpb, 0)
        pltpu.async_copy(y_hbm.at[gidx], rows, sem_r0)

    def fire_dchunk(ci, p):
        cc = lax.rem(ci + s * 10, NSCH)
        pltpu.async_copy(
            dst_hbm.at[c, pl.ds(pl.multiple_of(cc * SCH, SCH), SCH)],
            dchunk.at[p], sem_d)

    def wait_dchunk(p):
        pltpu.make_async_copy(dst_hbm.at[c, pl.ds(0, SCH)],
                              dchunk.at[p], sem_d).wait()

    def chunk(ci, carry):
        w, infl = carry
        cc = lax.rem(ci + s * 10, NSCH)
        p_ = ci & 1
        wait_dchunk(p_)

        @pl.when(ci < NSCH - 1)
        def _():
            fire_dchunk(ci + 1, 1 - p_)

        def scan4(v, w2):
            G = 4
            ds_ = [dchunk[p_, pl.ds((v * G + u) * 16, 16)] for u in range(G)]
            bases = [cc * SCH + (v * G + u) * 16 for u in range(G)]
            ns = [dv - coff for dv in ds_]
            ms = [((((nv * MAGIC) >> MSHIFT) == s) & (iota < (EP - b)))
                  for nv, b in zip(ns, bases)]
            cnts = [plsc.all_reduce_population_count(mv)[0] for mv in ms]
            offs = [w2]
            for u in range(G - 1):
                offs.append(offs[-1] + cnts[u])
            for u in range(G):
                plsc.store_compressed(elist.at[pl.ds(offs[u], 16)],
                                      ((bases[u] >> 1) + yhalf) + hiota,
                                      mask=ms[u])
                plsc.store_compressed(dlist.at[pl.ds(offs[u], 16)],
                                      ds_[u] + pvc, mask=ms[u])
            return offs[G - 1] + cnts[G - 1]
        w = lax.fori_loop(0, SCH // 64, scan4, w)

        @pl.when(infl == 1)
        def _():
            wait_async_batch()
            accum_from(gpack, rows, DB // 16)

        nb = w >> 7

        def dr(k, carry2):
            sync_drain(k * DB, DB // 16)
            return carry2
        lax.fori_loop(1, nb, dr, 0)

        @pl.when(nb >= 1)
        def _():
            fire_async_batch()
        rem = w & (DB - 1)
        nbdb = w - rem

        def cp(t, carry2):
            elist[pl.ds(t * 16, 16)] = elist[pl.ds(nbdb + t * 16, 16)]
            dlist[pl.ds(t * 16, 16)] = dlist[pl.ds(nbdb + t * 16, 16)]
            return carry2
        lax.fori_loop(0, DB // 16, cp, 0)
        return (rem, (nb >= 1).astype(jnp.int32))
    fire_dchunk(jnp.int32(0), 0)
    w, infl = lax.fori_loop(0, NSCH, chunk, (jnp.int32(0), jnp.int32(0)))

    @pl.when(infl == 1)
    def _():
        wait_async_batch()
        accum_from(gpack, rows, DB // 16)

    plsc.store_compressed(dlist.at[pl.ds(w, 16)],
                          jnp.full((16,), NPT, jnp.int32),
                          mask=jnp.full((16,), True))
    sync_drain(0, (w + 15) >> 4)

    pltpu.sync_copy(table.at[pl.ds(0, NPT)],
                    m_hbm.at[c, pl.ds(pl.multiple_of(nodebase, 128), NPT)])


def _segmax(y2, dst_adj):
    f = pl.kernel(
        _segmax_body,
        out_type=jax.ShapeDtypeStruct((2, NM, H), jnp.float32),
        mesh=_mesh,
        compiler_params=pltpu.CompilerParams(needs_layout_passes=False),
        scratch_types=[
            pltpu.VMEM((2, SCH), jnp.int32),
            pltpu.VMEM((ECAP,), jnp.int32),
            pltpu.VMEM((ECAP,), jnp.int32),
            pltpu.VMEM((DB,), jnp.int32),
            pltpu.VMEM((DB,), jnp.int32),
            pltpu.VMEM((DB, 2 * H), jnp.float32),
            pltpu.VMEM((NPT + 1, H), jnp.float32),
            pltpu.SemaphoreType.DMA,
            pltpu.SemaphoreType.DMA,
            pltpu.SemaphoreType.DMA,
        ],
    )
    return f(y2, dst_adj)


def kernel(x, batch, tpl_edge_index, euc_edge_index,
           W01, b01, g01, be01, W02, b02, g02, be02,
           W11, b11, g11, be11, W12, b12, g12, be12,
           Wm, bm, gm, bem):
    s = (1.0 / jnp.sqrt(jnp.float32(1.0 + EPS))).astype(jnp.float32)
    lo = jnp.arange(N, dtype=jnp.int32)
    zpad = jnp.zeros((EP_PAD - EP,), jnp.int32)

    def prep(ei, cidx):
        dst = jnp.concatenate([ei[1], lo, zpad]) + (cidx * N)
        src = jnp.concatenate([ei[0], lo, zpad]) + (cidx * N)
        return dst, src

    dst0, src0 = prep(tpl_edge_index, 0)
    dst1, src1 = prep(euc_edge_index, 1)
    dst_adj = jnp.stack([dst0, dst1])
    src_adj = jnp.stack([src0, src1])

    zb = jnp.zeros((H,), jnp.float32)
    wcat0 = jnp.concatenate([W01[:D] - W01[D:], W01[D:]], axis=1)
    wcat1 = jnp.concatenate([W11[:D] - W11[D:], W11[D:]], axis=1)
    wstack = jnp.stack([wcat0, wcat1])
    bstack = jnp.stack([jnp.concatenate([b01, zb]),
                        jnp.concatenate([b11, zb])])[:, None, :]

    eye2 = jnp.eye(2, dtype=jnp.float32)

    def fold2(g1, be1, W2, b2):
        w2f = (g1 * s)[:, None] * W2
        b2f = be1 @ W2 + b2
        return jnp.kron(eye2, w2f), jnp.tile(b2f, 2)

    w2big0, b2big0 = fold2(g01, be01, W02, b02)
    w2big1, b2big1 = fold2(g11, be11, W12, b12)
    w2bigs = jnp.stack([w2big0, w2big1]).astype(jnp.bfloat16)
    b2bigs = jnp.stack([b2big0, b2big1])[:, None, :]

    scat = jnp.concatenate([g02 * s, g12 * s])
    becat = jnp.concatenate([be02, be12])
    wmf = scat[:, None] * Wm
    bmf = (becat @ Wm + bm)[None]
    gms = (gm * s)[None]
    bem2 = bem[None]

    t = _node_mm(x, wstack, bstack)
    t_flat = t.reshape(2 * N, D)
    p2 = _edge_gather(t_flat, dst_adj, src_adj)
    y2 = _edge_mlp(p2, w2bigs, b2bigs)
    y_flat = y2.reshape(2 * EPH, 128)
    m = _segmax(y_flat, dst_adj)
    return _final_mlp(m, wmf, bmf, gms, bem2)
